# Initial kernel scaffold; baseline (speedup 1.0000x reference)
import jax
import jax.numpy as jnp
from jax.experimental import pallas as pl


def kernel(stu_id, input_exercise, input_knowledge_point, teacher_x, teacher_edge_index, teacher_batch, student_emb, k_lin_W, k_lin_b, k_conv1_W, k_conv1_b, k_conv2_W, k_conv2_b, k_fc_W, k_fc_b, e_lin_W, e_lin_b, e_conv1_W, e_conv1_b, e_conv2_W, e_conv2_b, e_fc_W, e_fc_b, pf1_W, pf1_b, pf2_W, pf2_b, pf3_W, pf3_b):
    raise NotImplementedError("write your pallas kernel here")


# speedup vs baseline: 31.8015x; 31.8015x over previous
import functools

import jax
import jax.numpy as jnp
from jax import lax
from jax.experimental import pallas as pl
from jax.experimental.pallas import tpu as pltpu
from jax.experimental.pallas import tpu_sc as plsc

N = 10000
E = 320000
B = 4096
K = 128
IN_FEAT = 4096
HID = 40
NACC = 10240
EPAD = 327680
NPADROWS = 240
NW = 32
RPT = NACC // 16

_MESH = dict(core_axis_name="c", subcore_axis_name="s")
_SC_PARAMS = pltpu.CompilerParams(use_tc_tiling_on_sc=False)


def _sc_degree(dst2d, ones_col, zeros_col):
    @functools.partial(
        pl.kernel,
        out_type=jax.ShapeDtypeStruct((2 * NACC, 8), jnp.float32),
        mesh=plsc.VectorSubcoreMesh(**_MESH),
        compiler_params=_SC_PARAMS,
        scratch_types=[
            pltpu.VMEM((8, 128), jnp.int32),
            pltpu.VMEM((128, 8), jnp.float32),
            pltpu.VMEM_SHARED((NACC, 8), jnp.float32),
            pltpu.SemaphoreType.DMA,
        ],
    )
    def deg_kernel(dst_h, ones_h, zeros_h, out_h, dst_v, ones_v, acc, sem):
        c = lax.axis_index("c")
        s = lax.axis_index("s")
        wid = s * 2 + c
        r0 = s * RPT
        pltpu.sync_copy(zeros_h.at[pl.ds(r0, RPT)], acc.at[pl.ds(r0, RPT)])
        pltpu.sync_copy(ones_h, ones_v)
        plsc.subcore_barrier()
        base = wid * 80

        def body(i, carry):
            row = base + i * 8
            pltpu.sync_copy(dst_h.at[pl.ds(row, 8)], dst_v)
            descs = [
                pltpu.async_copy(ones_v, acc.at[dst_v.at[j]], sem, add=True)
                for j in range(8)
            ]
            for d in descs:
                d.wait()
            return carry

        lax.fori_loop(0, 10, body, 0)
        plsc.subcore_barrier()
        pltpu.sync_copy(acc.at[pl.ds(r0, RPT)], out_h.at[pl.ds(c * NACC + r0, RPT)])

    return deg_kernel(dst2d, ones_col, zeros_col)


def _sc_aggregate(y, src2d, dst2d, zeros, w):
    @functools.partial(
        pl.kernel,
        out_type=jax.ShapeDtypeStruct((2 * NACC, w), jnp.float32),
        mesh=plsc.VectorSubcoreMesh(**_MESH),
        compiler_params=_SC_PARAMS,
        scratch_types=[
            pltpu.VMEM((8, 128), jnp.int32),
            pltpu.VMEM((8, 128), jnp.int32),
            pltpu.VMEM((8, 128, w), jnp.float32),
            pltpu.VMEM_SHARED((NACC, w), jnp.float32),
            pltpu.SemaphoreType.DMA,
            pltpu.SemaphoreType.DMA,
        ],
    )
    def agg_kernel(y_h, src_h, dst_h, zeros_h, out_h, src_v, dst_v, msg_v, acc,
                   gsem, ssem):
        c = lax.axis_index("c")
        s = lax.axis_index("s")
        wid = s * 2 + c
        r0 = s * RPT
        pltpu.sync_copy(zeros_h.at[pl.ds(r0, RPT)], acc.at[pl.ds(r0, RPT)])
        plsc.subcore_barrier()
        base = wid * 80

        def body(i, carry):
            row = base + i * 8
            pltpu.sync_copy(src_h.at[pl.ds(row, 8)], src_v)
            pltpu.sync_copy(dst_h.at[pl.ds(row, 8)], dst_v)
            gd = [
                pltpu.async_copy(y_h.at[src_v.at[j]], msg_v.at[j], gsem)
                for j in range(8)
            ]
            for d in gd:
                d.wait()
            sd = [
                pltpu.async_copy(msg_v.at[j], acc.at[dst_v.at[j]], ssem, add=True)
                for j in range(8)
            ]
            for d in sd:
                d.wait()
            return carry

        lax.fori_loop(0, 10, body, 0)
        plsc.subcore_barrier()
        pltpu.sync_copy(acc.at[pl.ds(r0, RPT)], out_h.at[pl.ds(c * NACC + r0, RPT)])

    return agg_kernel(y, src2d, dst2d, zeros)


def _sc_embed_gather(table, idx):
    bpw = B // NW

    @functools.partial(
        pl.kernel,
        out_type=jax.ShapeDtypeStruct((B, K), jnp.float32),
        mesh=plsc.VectorSubcoreMesh(**_MESH),
        compiler_params=_SC_PARAMS,
        scratch_types=[
            pltpu.VMEM((bpw,), jnp.int32),
            pltpu.VMEM((bpw, K), jnp.float32),
            pltpu.SemaphoreType.DMA,
        ],
    )
    def gather_kernel(table_h, idx_h, out_h, idx_v, rows_v, sem):
        c = lax.axis_index("c")
        s = lax.axis_index("s")
        wid = s * 2 + c
        base = wid * bpw
        pltpu.sync_copy(idx_h.at[pl.ds(base, bpw)], idx_v)
        pltpu.async_copy(table_h.at[idx_v], rows_v, sem).wait()
        pltpu.sync_copy(rows_v, out_h.at[pl.ds(base, bpw)])

    return gather_kernel(table, idx)


def _tc_big_matmul(x, wc, bc):
    BN = 1000

    def f(x_ref, w_ref, b_ref, o_ref):
        o_ref[...] = (
            jnp.dot(x_ref[...], w_ref[...], preferred_element_type=jnp.float32)
            + b_ref[...]
        )

    return pl.pallas_call(
        f,
        grid=(N // BN,),
        in_specs=[
            pl.BlockSpec((BN, IN_FEAT), lambda i: (i, 0)),
            pl.BlockSpec((IN_FEAT, 32), lambda i: (0, 0)),
            pl.BlockSpec((1, 32), lambda i: (0, 0)),
        ],
        out_specs=pl.BlockSpec((BN, 32), lambda i: (i, 0)),
        out_shape=jax.ShapeDtypeStruct((N, 32), jnp.float32),
    )(x, wc, bc)


def _tc_mid1(degp, h0):
    def f(degp_ref, h0_ref, dinv_ref, y0_ref):
        deg = (degp_ref[:NACC, :1] + degp_ref[NACC:, :1]) + 1.0
        dinv = lax.rsqrt(deg)
        dinv_ref[...] = dinv
        y0_ref[:N] = dinv[:N] * h0_ref[...]
        y0_ref[N:] = jnp.zeros((NACC - N, 32), jnp.float32)

    return pl.pallas_call(
        f,
        out_shape=(
            jax.ShapeDtypeStruct((NACC, 1), jnp.float32),
            jax.ShapeDtypeStruct((NACC, 32), jnp.float32),
        ),
    )(degp, h0)


def _tc_mid2(agg0, y0, dinv, w1blk, b1cat):
    def f(a_ref, y0_ref, dinv_ref, w_ref, b_ref, y1k_ref, y1e_ref):
        dinv = dinv_ref[:N]
        g0 = dinv * (a_ref[:N] + a_ref[NACC:NACC + N] + y0_ref[:N])
        h1 = jax.nn.relu(
            jnp.dot(g0, w_ref[...], preferred_element_type=jnp.float32) + b_ref[...]
        )
        y1 = dinv * h1
        y1k_ref[:N] = y1[:, :HID]
        y1k_ref[N:] = jnp.zeros((NACC - N, HID), jnp.float32)
        y1e_ref[:N] = y1[:, HID:]
        y1e_ref[N:] = jnp.zeros((NACC - N, HID), jnp.float32)

    return pl.pallas_call(
        f,
        out_shape=(
            jax.ShapeDtypeStruct((NACC, HID), jnp.float32),
            jax.ShapeDtypeStruct((NACC, HID), jnp.float32),
        ),
    )(agg0, y0, dinv, w1blk, b1cat)


def _tc_mid3(aggk, agge, y1k, y1e, dinv, kc2w, kc2b, ec2w, ec2b,
             kfc_w, kfc_b, efc_w, efc_b):
    def f(ak_ref, ae_ref, y1k_ref, y1e_ref, dinv_ref, kw2_ref, kb2_ref,
          ew2_ref, eb2_ref, kw_ref, kb_ref, ew_ref, eb_ref, kd_ref, ed_ref):
        dinv = dinv_ref[:N]
        g1k = dinv * (ak_ref[:N] + ak_ref[NACC:NACC + N] + y1k_ref[:N])
        g1e = dinv * (ae_ref[:N] + ae_ref[NACC:NACC + N] + y1e_ref[:N])
        h2k = jax.nn.relu(
            jnp.dot(g1k, kw2_ref[...], preferred_element_type=jnp.float32)
            + kb2_ref[...]
        )
        h2e = jax.nn.relu(
            jnp.dot(g1e, ew2_ref[...], preferred_element_type=jnp.float32)
            + eb2_ref[...]
        )
        pk = jnp.mean(h2k, axis=0, keepdims=True)
        pe = jnp.mean(h2e, axis=0, keepdims=True)
        kd_ref[...] = jax.nn.sigmoid(
            jnp.dot(pk, kw_ref[...], preferred_element_type=jnp.float32)
            + kb_ref[...]
        )
        ed_ref[...] = jax.nn.sigmoid(
            jnp.dot(pe, ew_ref[...], preferred_element_type=jnp.float32)
            + eb_ref[...]
        )

    return pl.pallas_call(
        f,
        out_shape=(
            jax.ShapeDtypeStruct((1, K), jnp.float32),
            jax.ShapeDtypeStruct((1, 1), jnp.float32),
        ),
    )(aggk, agge, y1k, y1e, dinv, kc2w, kc2b, ec2w, ec2b,
      kfc_w, kfc_b, efc_w, efc_b)


def _tc_prednet(se_raw, kd, ed, kp, w1, b1, w2, b2, w3, b3):
    BN = 1024

    def f(se_ref, kd_ref, ed_ref, kp_ref, w1_ref, b1_ref, w2_ref, b2_ref,
          w3_ref, b3_ref, o_ref):
        x = ed_ref[0, 0] * (jax.nn.sigmoid(se_ref[...]) - kd_ref[...]) * kp_ref[...]
        z = jax.nn.sigmoid(
            jnp.dot(x, jnp.abs(w1_ref[...]), preferred_element_type=jnp.float32)
            + b1_ref[...]
        )
        z = jax.nn.sigmoid(
            jnp.dot(z, jnp.abs(w2_ref[...]), preferred_element_type=jnp.float32)
            + b2_ref[...]
        )
        o_ref[...] = jax.nn.sigmoid(
            jnp.dot(z, jnp.abs(w3_ref[...]), preferred_element_type=jnp.float32)
            + b3_ref[...]
        )

    return pl.pallas_call(
        f,
        grid=(B // BN,),
        in_specs=[
            pl.BlockSpec((BN, K), lambda i: (i, 0)),
            pl.BlockSpec((1, K), lambda i: (0, 0)),
            pl.BlockSpec((1, 1), lambda i: (0, 0)),
            pl.BlockSpec((BN, K), lambda i: (i, 0)),
            pl.BlockSpec((K, 512), lambda i: (0, 0)),
            pl.BlockSpec((1, 512), lambda i: (0, 0)),
            pl.BlockSpec((512, 256), lambda i: (0, 0)),
            pl.BlockSpec((1, 256), lambda i: (0, 0)),
            pl.BlockSpec((256, 1), lambda i: (0, 0)),
            pl.BlockSpec((1, 1), lambda i: (0, 0)),
        ],
        out_specs=pl.BlockSpec((BN, 1), lambda i: (i, 0)),
        out_shape=jax.ShapeDtypeStruct((B, 1), jnp.float32),
    )(se_raw, kd, ed, kp, w1, b1, w2, b2, w3, b3)


def kernel(stu_id, input_exercise, input_knowledge_point, teacher_x,
           teacher_edge_index, teacher_batch, student_emb, k_lin_W, k_lin_b,
           k_conv1_W, k_conv1_b, k_conv2_W, k_conv2_b, k_fc_W, k_fc_b,
           e_lin_W, e_lin_b, e_conv1_W, e_conv1_b, e_conv2_W, e_conv2_b,
           e_fc_W, e_fc_b, pf1_W, pf1_b, pf2_W, pf2_b, pf3_W, pf3_b):
    src = teacher_edge_index[0]
    dst = teacher_edge_index[1]

    pad_idx = N + (jnp.arange(EPAD - E, dtype=src.dtype) % NPADROWS)
    src2d = jnp.concatenate([src, pad_idx]).reshape(EPAD // 128, 128)
    dst2d = jnp.concatenate([dst, pad_idx]).reshape(EPAD // 128, 128)

    zeros80 = jnp.zeros((NACC, 80), jnp.float32)
    zeros_col = jnp.zeros((NACC, 8), jnp.float32)
    ones_col = jnp.ones((128, 8), jnp.float32)

    wc = jnp.concatenate([k_lin_W, e_lin_W], axis=1)
    bc = jnp.concatenate([k_lin_b, e_lin_b]).reshape(1, 32)
    z1640 = jnp.zeros((16, 40), jnp.float32)
    w1blk = jnp.concatenate(
        [jnp.concatenate([k_conv1_W, z1640], axis=1),
         jnp.concatenate([z1640, e_conv1_W], axis=1)], axis=0)
    b1cat = jnp.concatenate([k_conv1_b, e_conv1_b]).reshape(1, 80)

    degp = _sc_degree(dst2d, ones_col, zeros_col)
    h0 = _tc_big_matmul(teacher_x, wc, bc)
    dinv, y0 = _tc_mid1(degp, h0)
    agg0 = _sc_aggregate(y0, src2d, dst2d, zeros80[:, :32], 32)
    y1k, y1e = _tc_mid2(agg0, y0, dinv, w1blk, b1cat)
    agg1k = _sc_aggregate(y1k, src2d, dst2d, zeros80[:, :HID], HID)
    agg1e = _sc_aggregate(y1e, src2d, dst2d, zeros80[:, :HID], HID)
    kd, ed = _tc_mid3(agg1k, agg1e, y1k, y1e, dinv,
                      k_conv2_W, k_conv2_b.reshape(1, HID),
                      e_conv2_W, e_conv2_b.reshape(1, HID),
                      k_fc_W, k_fc_b.reshape(1, 1 * K),
                      e_fc_W, e_fc_b.reshape(1, 1))
    se_raw = _sc_embed_gather(student_emb, stu_id)
    out = _tc_prednet(se_raw, kd, ed, input_knowledge_point,
                      pf1_W, pf1_b.reshape(1, 512),
                      pf2_W, pf2_b.reshape(1, 256),
                      pf3_W, pf3_b.reshape(1, 1))
    return out.reshape(-1)


# speedup vs baseline: 37.0897x; 1.1663x over previous
import functools

import jax
import jax.numpy as jnp
from jax import lax
from jax.experimental import pallas as pl
from jax.experimental.pallas import tpu as pltpu
from jax.experimental.pallas import tpu_sc as plsc

N = 10000
E = 320000
B = 4096
K = 128
IN_FEAT = 4096
HID = 40
NACC = 10240
EPAD = 327680
NPADROWS = 240
NW = 32
RPT = NACC // 16

_MESH = dict(core_axis_name="c", subcore_axis_name="s")
_SC_PARAMS = pltpu.CompilerParams(use_tc_tiling_on_sc=False)


def _sc_degree(dst2d, ones_col, zeros_col):
    @functools.partial(
        pl.kernel,
        out_type=jax.ShapeDtypeStruct((2 * NACC, 8), jnp.float32),
        mesh=plsc.VectorSubcoreMesh(**_MESH),
        compiler_params=_SC_PARAMS,
        scratch_types=[
            pltpu.VMEM((80, 128), jnp.int32),
            pltpu.VMEM((128, 8), jnp.float32),
            pltpu.VMEM_SHARED((NACC, 8), jnp.float32),
            pltpu.SemaphoreType.DMA,
        ],
    )
    def deg_kernel(dst_h, ones_h, zeros_h, out_h, dst_v, ones_v, acc, sem):
        c = lax.axis_index("c")
        s = lax.axis_index("s")
        wid = s * 2 + c
        r0 = s * RPT
        pltpu.sync_copy(zeros_h.at[pl.ds(r0, RPT)], acc.at[pl.ds(r0, RPT)])
        pltpu.sync_copy(ones_h, ones_v)
        pltpu.sync_copy(dst_h.at[pl.ds(wid * 80, 80)], dst_v)
        plsc.subcore_barrier()

        lag = 8
        sca = {}
        for g in range(80):
            if g >= lag:
                sca[g - lag].wait()
            sca[g] = pltpu.async_copy(ones_v, acc.at[dst_v.at[g]], sem, add=True)
        for g in range(80 - lag, 80):
            sca[g].wait()
        plsc.subcore_barrier()
        pltpu.sync_copy(acc.at[pl.ds(r0, RPT)], out_h.at[pl.ds(c * NACC + r0, RPT)])

    return deg_kernel(dst2d, ones_col, zeros_col)


def _sc_aggregate(y, src2d, dst2d, zeros, w):
    @functools.partial(
        pl.kernel,
        out_type=jax.ShapeDtypeStruct((2 * NACC, w), jnp.float32),
        mesh=plsc.VectorSubcoreMesh(**_MESH),
        compiler_params=_SC_PARAMS,
        scratch_types=[
            pltpu.VMEM((80, 128), jnp.int32),
            pltpu.VMEM((80, 128), jnp.int32),
            pltpu.VMEM((4, 128, w), jnp.float32),
            pltpu.VMEM_SHARED((NACC, w), jnp.float32),
            pltpu.SemaphoreType.DMA,
            pltpu.SemaphoreType.DMA,
        ],
    )
    def agg_kernel(y_h, src_h, dst_h, zeros_h, out_h, src_v, dst_v, msg_v, acc,
                   gsem, ssem):
        c = lax.axis_index("c")
        s = lax.axis_index("s")
        wid = s * 2 + c
        r0 = s * RPT
        pltpu.sync_copy(zeros_h.at[pl.ds(r0, RPT)], acc.at[pl.ds(r0, RPT)])
        pltpu.sync_copy(src_h.at[pl.ds(wid * 80, 80)], src_v)
        pltpu.sync_copy(dst_h.at[pl.ds(wid * 80, 80)], dst_v)
        plsc.subcore_barrier()

        gat, sca = {}, {}
        for g in range(3):
            gat[g] = pltpu.async_copy(y_h.at[src_v.at[g]], msg_v.at[g % 4], gsem)
        for g in range(80):
            if g >= 1:
                sca[g - 1].wait()
            if g + 3 < 80:
                gat[g + 3] = pltpu.async_copy(
                    y_h.at[src_v.at[g + 3]], msg_v.at[(g + 3) % 4], gsem)
            gat[g].wait()
            sca[g] = pltpu.async_copy(
                msg_v.at[g % 4], acc.at[dst_v.at[g]], ssem, add=True)
        sca[79].wait()
        plsc.subcore_barrier()
        pltpu.sync_copy(acc.at[pl.ds(r0, RPT)], out_h.at[pl.ds(c * NACC + r0, RPT)])

    return agg_kernel(y, src2d, dst2d, zeros)


def _sc_embed_gather(table, idx):
    bpw = B // NW

    @functools.partial(
        pl.kernel,
        out_type=jax.ShapeDtypeStruct((B, K), jnp.float32),
        mesh=plsc.VectorSubcoreMesh(**_MESH),
        compiler_params=_SC_PARAMS,
        scratch_types=[
            pltpu.VMEM((bpw,), jnp.int32),
            pltpu.VMEM((bpw, K), jnp.float32),
            pltpu.SemaphoreType.DMA,
        ],
    )
    def gather_kernel(table_h, idx_h, out_h, idx_v, rows_v, sem):
        c = lax.axis_index("c")
        s = lax.axis_index("s")
        wid = s * 2 + c
        base = wid * bpw
        pltpu.sync_copy(idx_h.at[pl.ds(base, bpw)], idx_v)
        pltpu.async_copy(table_h.at[idx_v], rows_v, sem).wait()
        pltpu.sync_copy(rows_v, out_h.at[pl.ds(base, bpw)])

    return gather_kernel(table, idx)


def _tc_big_matmul(x, wc, bc):
    BN = 1000

    def f(x_ref, w_ref, b_ref, o_ref):
        o_ref[...] = (
            jnp.dot(x_ref[...], w_ref[...], preferred_element_type=jnp.float32)
            + b_ref[...]
        )

    return pl.pallas_call(
        f,
        grid=(N // BN,),
        in_specs=[
            pl.BlockSpec((BN, IN_FEAT), lambda i: (i, 0)),
            pl.BlockSpec((IN_FEAT, 32), lambda i: (0, 0)),
            pl.BlockSpec((1, 32), lambda i: (0, 0)),
        ],
        out_specs=pl.BlockSpec((BN, 32), lambda i: (i, 0)),
        out_shape=jax.ShapeDtypeStruct((N, 32), jnp.float32),
    )(x, wc, bc)


def _tc_mid1(degp, h0):
    def f(degp_ref, h0_ref, dinv_ref, y0_ref):
        deg = (degp_ref[:NACC, :1] + degp_ref[NACC:, :1]) + 1.0
        dinv = lax.rsqrt(deg)
        dinv_ref[...] = dinv
        y0_ref[:N] = dinv[:N] * h0_ref[...]
        y0_ref[N:] = jnp.zeros((NACC - N, 32), jnp.float32)

    return pl.pallas_call(
        f,
        out_shape=(
            jax.ShapeDtypeStruct((NACC, 1), jnp.float32),
            jax.ShapeDtypeStruct((NACC, 32), jnp.float32),
        ),
    )(degp, h0)


def _tc_mid2(agg0, y0, dinv, w1blk, b1cat):
    def f(a_ref, y0_ref, dinv_ref, w_ref, b_ref, y1k_ref, y1e_ref):
        dinv = dinv_ref[:N]
        g0 = dinv * (a_ref[:N] + a_ref[NACC:NACC + N] + y0_ref[:N])
        h1 = jax.nn.relu(
            jnp.dot(g0, w_ref[...], preferred_element_type=jnp.float32) + b_ref[...]
        )
        y1 = dinv * h1
        y1k_ref[:N] = y1[:, :HID]
        y1k_ref[N:] = jnp.zeros((NACC - N, HID), jnp.float32)
        y1e_ref[:N] = y1[:, HID:]
        y1e_ref[N:] = jnp.zeros((NACC - N, HID), jnp.float32)

    return pl.pallas_call(
        f,
        out_shape=(
            jax.ShapeDtypeStruct((NACC, HID), jnp.float32),
            jax.ShapeDtypeStruct((NACC, HID), jnp.float32),
        ),
    )(agg0, y0, dinv, w1blk, b1cat)


def _tc_mid3(aggk, agge, y1k, y1e, dinv, kc2w, kc2b, ec2w, ec2b,
             kfc_w, kfc_b, efc_w, efc_b):
    def f(ak_ref, ae_ref, y1k_ref, y1e_ref, dinv_ref, kw2_ref, kb2_ref,
          ew2_ref, eb2_ref, kw_ref, kb_ref, ew_ref, eb_ref, kd_ref, ed_ref):
        dinv = dinv_ref[:N]
        g1k = dinv * (ak_ref[:N] + ak_ref[NACC:NACC + N] + y1k_ref[:N])
        g1e = dinv * (ae_ref[:N] + ae_ref[NACC:NACC + N] + y1e_ref[:N])
        h2k = jax.nn.relu(
            jnp.dot(g1k, kw2_ref[...], preferred_element_type=jnp.float32)
            + kb2_ref[...]
        )
        h2e = jax.nn.relu(
            jnp.dot(g1e, ew2_ref[...], preferred_element_type=jnp.float32)
            + eb2_ref[...]
        )
        pk = jnp.mean(h2k, axis=0, keepdims=True)
        pe = jnp.mean(h2e, axis=0, keepdims=True)
        kd_ref[...] = jax.nn.sigmoid(
            jnp.dot(pk, kw_ref[...], preferred_element_type=jnp.float32)
            + kb_ref[...]
        )
        ed_ref[...] = jax.nn.sigmoid(
            jnp.dot(pe, ew_ref[...], preferred_element_type=jnp.float32)
            + eb_ref[...]
        )

    return pl.pallas_call(
        f,
        out_shape=(
            jax.ShapeDtypeStruct((1, K), jnp.float32),
            jax.ShapeDtypeStruct((1, 1), jnp.float32),
        ),
    )(aggk, agge, y1k, y1e, dinv, kc2w, kc2b, ec2w, ec2b,
      kfc_w, kfc_b, efc_w, efc_b)


def _tc_prednet(se_raw, kd, ed, kp, w1, b1, w2, b2, w3, b3):
    BN = 1024

    def f(se_ref, kd_ref, ed_ref, kp_ref, w1_ref, b1_ref, w2_ref, b2_ref,
          w3_ref, b3_ref, o_ref):
        x = ed_ref[0, 0] * (jax.nn.sigmoid(se_ref[...]) - kd_ref[...]) * kp_ref[...]
        z = jax.nn.sigmoid(
            jnp.dot(x, jnp.abs(w1_ref[...]), preferred_element_type=jnp.float32)
            + b1_ref[...]
        )
        z = jax.nn.sigmoid(
            jnp.dot(z, jnp.abs(w2_ref[...]), preferred_element_type=jnp.float32)
            + b2_ref[...]
        )
        o_ref[...] = jax.nn.sigmoid(
            jnp.dot(z, jnp.abs(w3_ref[...]), preferred_element_type=jnp.float32)
            + b3_ref[...]
        )

    return pl.pallas_call(
        f,
        grid=(B // BN,),
        in_specs=[
            pl.BlockSpec((BN, K), lambda i: (i, 0)),
            pl.BlockSpec((1, K), lambda i: (0, 0)),
            pl.BlockSpec((1, 1), lambda i: (0, 0)),
            pl.BlockSpec((BN, K), lambda i: (i, 0)),
            pl.BlockSpec((K, 512), lambda i: (0, 0)),
            pl.BlockSpec((1, 512), lambda i: (0, 0)),
            pl.BlockSpec((512, 256), lambda i: (0, 0)),
            pl.BlockSpec((1, 256), lambda i: (0, 0)),
            pl.BlockSpec((256, 1), lambda i: (0, 0)),
            pl.BlockSpec((1, 1), lambda i: (0, 0)),
        ],
        out_specs=pl.BlockSpec((BN, 1), lambda i: (i, 0)),
        out_shape=jax.ShapeDtypeStruct((B, 1), jnp.float32),
    )(se_raw, kd, ed, kp, w1, b1, w2, b2, w3, b3)


def kernel(stu_id, input_exercise, input_knowledge_point, teacher_x,
           teacher_edge_index, teacher_batch, student_emb, k_lin_W, k_lin_b,
           k_conv1_W, k_conv1_b, k_conv2_W, k_conv2_b, k_fc_W, k_fc_b,
           e_lin_W, e_lin_b, e_conv1_W, e_conv1_b, e_conv2_W, e_conv2_b,
           e_fc_W, e_fc_b, pf1_W, pf1_b, pf2_W, pf2_b, pf3_W, pf3_b):
    src = teacher_edge_index[0]
    dst = teacher_edge_index[1]

    pad_idx = N + (jnp.arange(EPAD - E, dtype=src.dtype) % NPADROWS)
    src2d = jnp.concatenate([src, pad_idx]).reshape(EPAD // 128, 128)
    dst2d = jnp.concatenate([dst, pad_idx]).reshape(EPAD // 128, 128)

    zeros80 = jnp.zeros((NACC, 80), jnp.float32)
    zeros_col = jnp.zeros((NACC, 8), jnp.float32)
    ones_col = jnp.ones((128, 8), jnp.float32)

    wc = jnp.concatenate([k_lin_W, e_lin_W], axis=1)
    bc = jnp.concatenate([k_lin_b, e_lin_b]).reshape(1, 32)
    z1640 = jnp.zeros((16, 40), jnp.float32)
    w1blk = jnp.concatenate(
        [jnp.concatenate([k_conv1_W, z1640], axis=1),
         jnp.concatenate([z1640, e_conv1_W], axis=1)], axis=0)
    b1cat = jnp.concatenate([k_conv1_b, e_conv1_b]).reshape(1, 80)

    degp = _sc_degree(dst2d, ones_col, zeros_col)
    h0 = _tc_big_matmul(teacher_x, wc, bc)
    dinv, y0 = _tc_mid1(degp, h0)
    agg0 = _sc_aggregate(y0, src2d, dst2d, zeros80[:, :32], 32)
    y1k, y1e = _tc_mid2(agg0, y0, dinv, w1blk, b1cat)
    agg1k = _sc_aggregate(y1k, src2d, dst2d, zeros80[:, :HID], HID)
    agg1e = _sc_aggregate(y1e, src2d, dst2d, zeros80[:, :HID], HID)
    kd, ed = _tc_mid3(agg1k, agg1e, y1k, y1e, dinv,
                      k_conv2_W, k_conv2_b.reshape(1, HID),
                      e_conv2_W, e_conv2_b.reshape(1, HID),
                      k_fc_W, k_fc_b.reshape(1, 1 * K),
                      e_fc_W, e_fc_b.reshape(1, 1))
    se_raw = _sc_embed_gather(student_emb, stu_id)
    out = _tc_prednet(se_raw, kd, ed, input_knowledge_point,
                      pf1_W, pf1_b.reshape(1, 512),
                      pf2_W, pf2_b.reshape(1, 256),
                      pf3_W, pf3_b.reshape(1, 1))
    return out.reshape(-1)


# speedup vs baseline: 39.5989x; 1.0677x over previous
import functools

import numpy as np

import jax
import jax.numpy as jnp
from jax import lax
from jax.experimental import pallas as pl
from jax.experimental.pallas import tpu as pltpu
from jax.experimental.pallas import tpu_sc as plsc

N = 10000
E = 320000
B = 4096
K = 128
IN_FEAT = 4096
HID = 40
NACC = 10240
EPAD = 327680
NPADROWS = 240
NW = 32
RPT = NACC // 16

_MESH = dict(core_axis_name="c", subcore_axis_name="s")
_SC_PARAMS = pltpu.CompilerParams(use_tc_tiling_on_sc=False)

_PAD_ROWS = np.asarray(
    N + (np.arange(60 * 128, dtype=np.int32) % NPADROWS)).reshape(60, 128)


def _sc_degree(dst2d, ones_col, zeros_col):
    @functools.partial(
        pl.kernel,
        out_type=jax.ShapeDtypeStruct((2 * NACC, 8), jnp.float32),
        mesh=plsc.VectorSubcoreMesh(**_MESH),
        compiler_params=_SC_PARAMS,
        scratch_types=[
            pltpu.VMEM((80, 128), jnp.int32),
            pltpu.VMEM((128, 8), jnp.float32),
            pltpu.VMEM_SHARED((NACC, 8), jnp.float32),
            pltpu.SemaphoreType.DMA,
        ],
    )
    def deg_kernel(dst_h, ones_h, zeros_h, out_h, dst_v, ones_v, acc, sem):
        c = lax.axis_index("c")
        s = lax.axis_index("s")
        wid = s * 2 + c
        r0 = s * RPT
        pltpu.sync_copy(zeros_h.at[pl.ds(r0, RPT)], acc.at[pl.ds(r0, RPT)])
        pltpu.sync_copy(ones_h, ones_v)
        pltpu.sync_copy(dst_h.at[pl.ds(wid * 80, 80)], dst_v)
        plsc.subcore_barrier()

        lag = 8
        sca = {}
        for g in range(80):
            if g >= lag:
                sca[g - lag].wait()
            sca[g] = pltpu.async_copy(ones_v, acc.at[dst_v.at[g]], sem, add=True)
        for g in range(80 - lag, 80):
            sca[g].wait()
        plsc.subcore_barrier()
        pltpu.sync_copy(acc.at[pl.ds(r0, RPT)], out_h.at[pl.ds(c * NACC + r0, RPT)])

    return deg_kernel(dst2d, ones_col, zeros_col)


def _sc_aggregate(y, src2d, dst2d, zeros, w):
    @functools.partial(
        pl.kernel,
        out_type=jax.ShapeDtypeStruct((2 * NACC, w), jnp.float32),
        mesh=plsc.VectorSubcoreMesh(**_MESH),
        compiler_params=_SC_PARAMS,
        scratch_types=[
            pltpu.VMEM((80, 128), jnp.int32),
            pltpu.VMEM((80, 128), jnp.int32),
            pltpu.VMEM((4, 128, w), jnp.float32),
            pltpu.VMEM_SHARED((NACC, w), jnp.float32),
            pltpu.SemaphoreType.DMA,
            pltpu.SemaphoreType.DMA,
        ],
    )
    def agg_kernel(y_h, src_h, dst_h, zeros_h, out_h, src_v, dst_v, msg_v, acc,
                   gsem, ssem):
        c = lax.axis_index("c")
        s = lax.axis_index("s")
        wid = s * 2 + c
        r0 = s * RPT
        pltpu.sync_copy(zeros_h.at[pl.ds(r0, RPT)], acc.at[pl.ds(r0, RPT)])
        pltpu.sync_copy(src_h.at[pl.ds(wid * 80, 80)], src_v)
        pltpu.sync_copy(dst_h.at[pl.ds(wid * 80, 80)], dst_v)
        plsc.subcore_barrier()

        gat, sca = {}, {}
        for g in range(3):
            gat[g] = pltpu.async_copy(y_h.at[src_v.at[g]], msg_v.at[g % 4], gsem)
        for g in range(80):
            if g >= 1:
                sca[g - 1].wait()
            if g + 3 < 80:
                gat[g + 3] = pltpu.async_copy(
                    y_h.at[src_v.at[g + 3]], msg_v.at[(g + 3) % 4], gsem)
            gat[g].wait()
            sca[g] = pltpu.async_copy(
                msg_v.at[g % 4], acc.at[dst_v.at[g]], ssem, add=True)
        sca[79].wait()
        plsc.subcore_barrier()
        pltpu.sync_copy(acc.at[pl.ds(r0, RPT)], out_h.at[pl.ds(c * NACC + r0, RPT)])

    return agg_kernel(y, src2d, dst2d, zeros)


def _sc_aggregate_split(ycat, srcb, dst2d, zeros):
    @functools.partial(
        pl.kernel,
        out_type=jax.ShapeDtypeStruct((2 * NACC, HID), jnp.float32),
        mesh=plsc.VectorSubcoreMesh(**_MESH),
        compiler_params=_SC_PARAMS,
        scratch_types=[
            pltpu.VMEM((80, 128), jnp.int32),
            pltpu.VMEM((80, 128), jnp.int32),
            pltpu.VMEM((4, 128, HID), jnp.float32),
            pltpu.VMEM_SHARED((NACC, HID), jnp.float32),
            pltpu.SemaphoreType.DMA,
            pltpu.SemaphoreType.DMA,
        ],
    )
    def agg2_kernel(y_h, src_h, dst_h, zeros_h, out_h, src_v, dst_v, msg_v,
                    acc, gsem, ssem):
        c = lax.axis_index("c")
        s = lax.axis_index("s")
        r0 = s * RPT
        pltpu.sync_copy(zeros_h.at[pl.ds(r0, RPT)], acc.at[pl.ds(r0, RPT)])
        plsc.subcore_barrier()

        def one_pass(p, carry):
            srow = c * 2560 + s * 160 + p * 80
            drow = s * 160 + p * 80
            pltpu.sync_copy(src_h.at[pl.ds(srow, 80)], src_v)
            pltpu.sync_copy(dst_h.at[pl.ds(drow, 80)], dst_v)
            gat, sca = {}, {}
            for g in range(3):
                gat[g] = pltpu.async_copy(
                    y_h.at[src_v.at[g]], msg_v.at[g % 4], gsem)
            for g in range(80):
                if g >= 1:
                    sca[g - 1].wait()
                if g + 3 < 80:
                    gat[g + 3] = pltpu.async_copy(
                        y_h.at[src_v.at[g + 3]], msg_v.at[(g + 3) % 4], gsem)
                gat[g].wait()
                sca[g] = pltpu.async_copy(
                    msg_v.at[g % 4], acc.at[dst_v.at[g]], ssem, add=True)
            sca[79].wait()
            return carry

        lax.fori_loop(0, 2, one_pass, 0)
        plsc.subcore_barrier()
        pltpu.sync_copy(acc.at[pl.ds(r0, RPT)], out_h.at[pl.ds(c * NACC + r0, RPT)])

    return agg2_kernel(ycat, srcb, dst2d, zeros)


def _sc_embed_gather(table, idx):
    bpw = B // NW

    @functools.partial(
        pl.kernel,
        out_type=jax.ShapeDtypeStruct((B, K), jnp.float32),
        mesh=plsc.VectorSubcoreMesh(**_MESH),
        compiler_params=_SC_PARAMS,
        scratch_types=[
            pltpu.VMEM((bpw,), jnp.int32),
            pltpu.VMEM((bpw, K), jnp.float32),
            pltpu.SemaphoreType.DMA,
        ],
    )
    def gather_kernel(table_h, idx_h, out_h, idx_v, rows_v, sem):
        c = lax.axis_index("c")
        s = lax.axis_index("s")
        wid = s * 2 + c
        base = wid * bpw
        pltpu.sync_copy(idx_h.at[pl.ds(base, bpw)], idx_v)
        pltpu.async_copy(table_h.at[idx_v], rows_v, sem).wait()
        pltpu.sync_copy(rows_v, out_h.at[pl.ds(base, bpw)])

    return gather_kernel(table, idx)


def _tc_big_matmul(x, wc, bc):
    BN = 1000

    def f(x_ref, w_ref, b_ref, o_ref):
        o_ref[...] = (
            jnp.dot(x_ref[...], w_ref[...], preferred_element_type=jnp.float32)
            + b_ref[...]
        )

    return pl.pallas_call(
        f,
        grid=(N // BN,),
        in_specs=[
            pl.BlockSpec((BN, IN_FEAT), lambda i: (i, 0)),
            pl.BlockSpec((IN_FEAT, 32), lambda i: (0, 0)),
            pl.BlockSpec((1, 32), lambda i: (0, 0)),
        ],
        out_specs=pl.BlockSpec((BN, 32), lambda i: (i, 0)),
        out_shape=jax.ShapeDtypeStruct((N, 32), jnp.float32),
    )(x, wc, bc)


def _tc_mid1(degp, h0):
    def f(degp_ref, h0_ref, dinv_ref, y0_ref):
        deg = (degp_ref[:NACC, :1] + degp_ref[NACC:, :1]) + 1.0
        dinv = lax.rsqrt(deg)
        dinv_ref[...] = dinv
        y0_ref[:N] = dinv[:N] * h0_ref[...]
        y0_ref[N:] = jnp.zeros((NACC - N, 32), jnp.float32)

    return pl.pallas_call(
        f,
        out_shape=(
            jax.ShapeDtypeStruct((NACC, 1), jnp.float32),
            jax.ShapeDtypeStruct((NACC, 32), jnp.float32),
        ),
    )(degp, h0)


def _tc_mid2(agg0, y0, dinv, w1blk, b1cat):
    def f(a_ref, y0_ref, dinv_ref, w_ref, b_ref, ycat_ref):
        dinv = dinv_ref[:N]
        g0 = dinv * (a_ref[:N] + a_ref[NACC:NACC + N] + y0_ref[:N])
        h1 = jax.nn.relu(
            jnp.dot(g0, w_ref[...], preferred_element_type=jnp.float32) + b_ref[...]
        )
        y1 = dinv * h1
        zpad = jnp.zeros((NACC - N, HID), jnp.float32)
        ycat_ref[:N] = y1[:, :HID]
        ycat_ref[N:NACC] = zpad
        ycat_ref[NACC:NACC + N] = y1[:, HID:]
        ycat_ref[NACC + N:] = zpad

    return pl.pallas_call(
        f,
        out_shape=jax.ShapeDtypeStruct((2 * NACC, HID), jnp.float32),
    )(agg0, y0, dinv, w1blk, b1cat)


def _tc_mid3(agg2, ycat, dinv, kc2w, kc2b, ec2w, ec2b,
             kfc_w, kfc_b, efc_w, efc_b):
    def f(a_ref, ycat_ref, dinv_ref, kw2_ref, kb2_ref,
          ew2_ref, eb2_ref, kw_ref, kb_ref, ew_ref, eb_ref, kd_ref, ed_ref):
        dinv = dinv_ref[:N]
        g1k = dinv * (a_ref[:N] + ycat_ref[:N])
        g1e = dinv * (a_ref[NACC:NACC + N] + ycat_ref[NACC:NACC + N])
        h2k = jax.nn.relu(
            jnp.dot(g1k, kw2_ref[...], preferred_element_type=jnp.float32)
            + kb2_ref[...]
        )
        h2e = jax.nn.relu(
            jnp.dot(g1e, ew2_ref[...], preferred_element_type=jnp.float32)
            + eb2_ref[...]
        )
        pk = jnp.mean(h2k, axis=0, keepdims=True)
        pe = jnp.mean(h2e, axis=0, keepdims=True)
        kd_ref[...] = jax.nn.sigmoid(
            jnp.dot(pk, kw_ref[...], preferred_element_type=jnp.float32)
            + kb_ref[...]
        )
        ed_ref[...] = jax.nn.sigmoid(
            jnp.dot(pe, ew_ref[...], preferred_element_type=jnp.float32)
            + eb_ref[...]
        )

    return pl.pallas_call(
        f,
        out_shape=(
            jax.ShapeDtypeStruct((1, K), jnp.float32),
            jax.ShapeDtypeStruct((1, 1), jnp.float32),
        ),
    )(agg2, ycat, dinv, kc2w, kc2b, ec2w, ec2b,
      kfc_w, kfc_b, efc_w, efc_b)


def _tc_prednet(se_raw, kd, ed, kp, w1, b1, w2, b2, w3, b3):
    BN = 1024

    def f(se_ref, kd_ref, ed_ref, kp_ref, w1_ref, b1_ref, w2_ref, b2_ref,
          w3_ref, b3_ref, o_ref):
        x = ed_ref[0, 0] * (jax.nn.sigmoid(se_ref[...]) - kd_ref[...]) * kp_ref[...]
        z = jax.nn.sigmoid(
            jnp.dot(x, jnp.abs(w1_ref[...]), preferred_element_type=jnp.float32)
            + b1_ref[...]
        )
        z = jax.nn.sigmoid(
            jnp.dot(z, jnp.abs(w2_ref[...]), preferred_element_type=jnp.float32)
            + b2_ref[...]
        )
        o_ref[...] = jax.nn.sigmoid(
            jnp.dot(z, jnp.abs(w3_ref[...]), preferred_element_type=jnp.float32)
            + b3_ref[...]
        )

    return pl.pallas_call(
        f,
        grid=(B // BN,),
        in_specs=[
            pl.BlockSpec((BN, K), lambda i: (i, 0)),
            pl.BlockSpec((1, K), lambda i: (0, 0)),
            pl.BlockSpec((1, 1), lambda i: (0, 0)),
            pl.BlockSpec((BN, K), lambda i: (i, 0)),
            pl.BlockSpec((K, 512), lambda i: (0, 0)),
            pl.BlockSpec((1, 512), lambda i: (0, 0)),
            pl.BlockSpec((512, 256), lambda i: (0, 0)),
            pl.BlockSpec((1, 256), lambda i: (0, 0)),
            pl.BlockSpec((256, 1), lambda i: (0, 0)),
            pl.BlockSpec((1, 1), lambda i: (0, 0)),
        ],
        out_specs=pl.BlockSpec((BN, 1), lambda i: (i, 0)),
        out_shape=jax.ShapeDtypeStruct((B, 1), jnp.float32),
    )(se_raw, kd, ed, kp, w1, b1, w2, b2, w3, b3)


def kernel(stu_id, input_exercise, input_knowledge_point, teacher_x,
           teacher_edge_index, teacher_batch, student_emb, k_lin_W, k_lin_b,
           k_conv1_W, k_conv1_b, k_conv2_W, k_conv2_b, k_fc_W, k_fc_b,
           e_lin_W, e_lin_b, e_conv1_W, e_conv1_b, e_conv2_W, e_conv2_b,
           e_fc_W, e_fc_b, pf1_W, pf1_b, pf2_W, pf2_b, pf3_W, pf3_b):
    src = teacher_edge_index[0]
    dst = teacher_edge_index[1]

    pad_rows = jnp.asarray(_PAD_ROWS)
    src2d = jnp.concatenate([src.reshape(E // 128, 128), pad_rows])
    dst2d = jnp.concatenate([dst.reshape(E // 128, 128), pad_rows])
    srcb = jnp.concatenate([src2d, src2d + NACC])

    zeros80 = jnp.zeros((NACC, 80), jnp.float32)
    zeros_col = jnp.zeros((NACC, 8), jnp.float32)
    ones_col = jnp.ones((128, 8), jnp.float32)

    wc = jnp.concatenate([k_lin_W, e_lin_W], axis=1)
    bc = jnp.concatenate([k_lin_b, e_lin_b]).reshape(1, 32)
    z1640 = jnp.zeros((16, 40), jnp.float32)
    w1blk = jnp.concatenate(
        [jnp.concatenate([k_conv1_W, z1640], axis=1),
         jnp.concatenate([z1640, e_conv1_W], axis=1)], axis=0)
    b1cat = jnp.concatenate([k_conv1_b, e_conv1_b]).reshape(1, 80)

    se_raw = _sc_embed_gather(student_emb, stu_id)
    degp = _sc_degree(dst2d, ones_col, zeros_col)
    h0 = _tc_big_matmul(teacher_x, wc, bc)
    dinv, y0 = _tc_mid1(degp, h0)
    agg0 = _sc_aggregate(y0, src2d, dst2d, zeros80[:, :32], 32)
    ycat = _tc_mid2(agg0, y0, dinv, w1blk, b1cat)
    agg2 = _sc_aggregate_split(ycat, srcb, dst2d, zeros80[:, :HID])
    kd, ed = _tc_mid3(agg2, ycat, dinv,
                      k_conv2_W, k_conv2_b.reshape(1, HID),
                      e_conv2_W, e_conv2_b.reshape(1, HID),
                      k_fc_W, k_fc_b.reshape(1, 1 * K),
                      e_fc_W, e_fc_b.reshape(1, 1))
    out = _tc_prednet(se_raw, kd, ed, input_knowledge_point,
                      pf1_W, pf1_b.reshape(1, 512),
                      pf2_W, pf2_b.reshape(1, 256),
                      pf3_W, pf3_b.reshape(1, 1))
    return out.reshape(-1)


# speedup vs baseline: 41.4186x; 1.0460x over previous
import functools

import numpy as np

import jax
import jax.numpy as jnp
from jax import lax
from jax.experimental import pallas as pl
from jax.experimental.pallas import tpu as pltpu
from jax.experimental.pallas import tpu_sc as plsc

N = 10000
E = 320000
B = 4096
K = 128
IN_FEAT = 4096
HID = 40
NACC = 10240
EPAD = 327680
NPADROWS = 240
NW = 32
RPT = NACC // 16

_MESH = dict(core_axis_name="c", subcore_axis_name="s")
_SC_PARAMS = pltpu.CompilerParams(use_tc_tiling_on_sc=False)

_PAD_ROWS = np.asarray(
    N + (np.arange(60 * 128, dtype=np.int32) % NPADROWS)).reshape(60, 128)


def _sc_degree(dst2d, ones_col, zeros_col):
    @functools.partial(
        pl.kernel,
        out_type=jax.ShapeDtypeStruct((2 * NACC, 8), jnp.float32),
        mesh=plsc.VectorSubcoreMesh(**_MESH),
        compiler_params=_SC_PARAMS,
        scratch_types=[
            pltpu.VMEM((80, 128), jnp.int32),
            pltpu.VMEM((128, 8), jnp.float32),
            pltpu.VMEM_SHARED((NACC, 8), jnp.float32),
            pltpu.SemaphoreType.DMA,
        ],
    )
    def deg_kernel(dst_h, ones_h, zeros_h, out_h, dst_v, ones_v, acc, sem):
        c = lax.axis_index("c")
        s = lax.axis_index("s")
        wid = s * 2 + c
        r0 = s * RPT
        pltpu.sync_copy(zeros_h.at[pl.ds(r0, RPT)], acc.at[pl.ds(r0, RPT)])
        pltpu.sync_copy(ones_h, ones_v)
        pltpu.sync_copy(dst_h.at[pl.ds(wid * 80, 80)], dst_v)
        plsc.subcore_barrier()

        lag = 8
        sca = {}
        for g in range(80):
            if g >= lag:
                sca[g - lag].wait()
            sca[g] = pltpu.async_copy(ones_v, acc.at[dst_v.at[g]], sem, add=True)
        for g in range(80 - lag, 80):
            sca[g].wait()
        plsc.subcore_barrier()
        pltpu.sync_copy(acc.at[pl.ds(r0, RPT)], out_h.at[pl.ds(c * NACC + r0, RPT)])

    return deg_kernel(dst2d, ones_col, zeros_col)


def _sc_aggregate(y, src2d, dst2d, zeros, w, dtype=jnp.bfloat16):
    @functools.partial(
        pl.kernel,
        out_type=jax.ShapeDtypeStruct((2 * NACC, w), dtype),
        mesh=plsc.VectorSubcoreMesh(**_MESH),
        compiler_params=_SC_PARAMS,
        scratch_types=[
            pltpu.VMEM((80, 128), jnp.int32),
            pltpu.VMEM((80, 128), jnp.int32),
            pltpu.VMEM((4, 128, w), dtype),
            pltpu.VMEM_SHARED((NACC, w), dtype),
            pltpu.SemaphoreType.DMA,
            pltpu.SemaphoreType.DMA,
        ],
    )
    def agg_kernel(y_h, src_h, dst_h, zeros_h, out_h, src_v, dst_v, msg_v, acc,
                   gsem, ssem):
        c = lax.axis_index("c")
        s = lax.axis_index("s")
        wid = s * 2 + c
        r0 = s * RPT
        pltpu.sync_copy(zeros_h.at[pl.ds(r0, RPT)], acc.at[pl.ds(r0, RPT)])
        pltpu.sync_copy(src_h.at[pl.ds(wid * 80, 80)], src_v)
        pltpu.sync_copy(dst_h.at[pl.ds(wid * 80, 80)], dst_v)
        plsc.subcore_barrier()

        gat, sca = {}, {}
        for g in range(3):
            gat[g] = pltpu.async_copy(y_h.at[src_v.at[g]], msg_v.at[g % 4], gsem)
        for g in range(80):
            if g >= 1:
                sca[g - 1].wait()
            if g + 3 < 80:
                gat[g + 3] = pltpu.async_copy(
                    y_h.at[src_v.at[g + 3]], msg_v.at[(g + 3) % 4], gsem)
            gat[g].wait()
            sca[g] = pltpu.async_copy(
                msg_v.at[g % 4], acc.at[dst_v.at[g]], ssem, add=True)
        sca[79].wait()
        plsc.subcore_barrier()
        pltpu.sync_copy(acc.at[pl.ds(r0, RPT)], out_h.at[pl.ds(c * NACC + r0, RPT)])

    return agg_kernel(y, src2d, dst2d, zeros)


def _sc_aggregate_split(ycat, srcb, dst2d, zeros):
    @functools.partial(
        pl.kernel,
        out_type=jax.ShapeDtypeStruct((2 * NACC, 48), jnp.bfloat16),
        mesh=plsc.VectorSubcoreMesh(**_MESH),
        compiler_params=_SC_PARAMS,
        scratch_types=[
            pltpu.VMEM((80, 128), jnp.int32),
            pltpu.VMEM((80, 128), jnp.int32),
            pltpu.VMEM((4, 128, 48), jnp.bfloat16),
            pltpu.VMEM_SHARED((NACC, 48), jnp.bfloat16),
            pltpu.SemaphoreType.DMA,
            pltpu.SemaphoreType.DMA,
        ],
    )
    def agg2_kernel(y_h, src_h, dst_h, zeros_h, out_h, src_v, dst_v, msg_v,
                    acc, gsem, ssem):
        c = lax.axis_index("c")
        s = lax.axis_index("s")
        r0 = s * RPT
        pltpu.sync_copy(zeros_h.at[pl.ds(r0, RPT)], acc.at[pl.ds(r0, RPT)])
        plsc.subcore_barrier()

        def one_pass(p, carry):
            srow = c * 2560 + s * 160 + p * 80
            drow = s * 160 + p * 80
            pltpu.sync_copy(src_h.at[pl.ds(srow, 80)], src_v)
            pltpu.sync_copy(dst_h.at[pl.ds(drow, 80)], dst_v)
            gat, sca = {}, {}
            for g in range(3):
                gat[g] = pltpu.async_copy(
                    y_h.at[src_v.at[g]], msg_v.at[g % 4], gsem)
            for g in range(80):
                if g >= 1:
                    sca[g - 1].wait()
                if g + 3 < 80:
                    gat[g + 3] = pltpu.async_copy(
                        y_h.at[src_v.at[g + 3]], msg_v.at[(g + 3) % 4], gsem)
                gat[g].wait()
                sca[g] = pltpu.async_copy(
                    msg_v.at[g % 4], acc.at[dst_v.at[g]], ssem, add=True)
            sca[79].wait()
            return carry

        lax.fori_loop(0, 2, one_pass, 0)
        plsc.subcore_barrier()
        pltpu.sync_copy(acc.at[pl.ds(r0, RPT)], out_h.at[pl.ds(c * NACC + r0, RPT)])

    return agg2_kernel(ycat, srcb, dst2d, zeros)


def _sc_embed_gather(table, idx):
    bpw = B // NW

    @functools.partial(
        pl.kernel,
        out_type=jax.ShapeDtypeStruct((B, K), jnp.float32),
        mesh=plsc.VectorSubcoreMesh(**_MESH),
        compiler_params=_SC_PARAMS,
        scratch_types=[
            pltpu.VMEM((bpw,), jnp.int32),
            pltpu.VMEM((bpw, K), jnp.float32),
            pltpu.SemaphoreType.DMA,
        ],
    )
    def gather_kernel(table_h, idx_h, out_h, idx_v, rows_v, sem):
        c = lax.axis_index("c")
        s = lax.axis_index("s")
        wid = s * 2 + c
        base = wid * bpw
        pltpu.sync_copy(idx_h.at[pl.ds(base, bpw)], idx_v)
        pltpu.async_copy(table_h.at[idx_v], rows_v, sem).wait()
        pltpu.sync_copy(rows_v, out_h.at[pl.ds(base, bpw)])

    return gather_kernel(table, idx)


def _tc_big_matmul(x, wc, bc):
    BN = 1000

    def f(x_ref, w_ref, b_ref, o_ref):
        o_ref[...] = (
            jnp.dot(x_ref[...], w_ref[...], preferred_element_type=jnp.float32)
            + b_ref[...]
        )

    return pl.pallas_call(
        f,
        grid=(N // BN,),
        in_specs=[
            pl.BlockSpec((BN, IN_FEAT), lambda i: (i, 0)),
            pl.BlockSpec((IN_FEAT, 32), lambda i: (0, 0)),
            pl.BlockSpec((1, 32), lambda i: (0, 0)),
        ],
        out_specs=pl.BlockSpec((BN, 32), lambda i: (i, 0)),
        out_shape=jax.ShapeDtypeStruct((N, 32), jnp.float32),
    )(x, wc, bc)


def _tc_mid1(degp, h0):
    def f(degp_ref, h0_ref, dinv_ref, y0_ref):
        deg = (degp_ref[:NACC, :1] + degp_ref[NACC:, :1]) + 1.0
        dinv = lax.rsqrt(deg)
        dinv_ref[...] = dinv
        y0_ref[:N] = (dinv[:N] * h0_ref[...]).astype(jnp.bfloat16)
        y0_ref[N:] = jnp.zeros((NACC - N, 32), jnp.bfloat16)

    return pl.pallas_call(
        f,
        out_shape=(
            jax.ShapeDtypeStruct((NACC, 1), jnp.float32),
            jax.ShapeDtypeStruct((NACC, 32), jnp.bfloat16),
        ),
    )(degp, h0)


def _tc_mid2(agg0, y0, dinv, w1blk, b1cat):
    def f(a_ref, y0_ref, dinv_ref, w_ref, b_ref, ycat_ref):
        dinv = dinv_ref[:N]
        asum = (a_ref[:N].astype(jnp.float32)
                + a_ref[NACC:NACC + N].astype(jnp.float32)
                + y0_ref[:N].astype(jnp.float32))
        g0 = dinv * asum
        h1 = jax.nn.relu(
            jnp.dot(g0, w_ref[...], preferred_element_type=jnp.float32) + b_ref[...]
        )
        y1 = dinv * h1
        cpad = jnp.zeros((N, 8), jnp.float32)
        zpad = jnp.zeros((NACC - N, 48), jnp.bfloat16)
        ycat_ref[:N] = jnp.concatenate(
            [y1[:, :HID], cpad], axis=1).astype(jnp.bfloat16)
        ycat_ref[N:NACC] = zpad
        ycat_ref[NACC:NACC + N] = jnp.concatenate(
            [y1[:, HID:], cpad], axis=1).astype(jnp.bfloat16)
        ycat_ref[NACC + N:] = zpad

    return pl.pallas_call(
        f,
        out_shape=jax.ShapeDtypeStruct((2 * NACC, 48), jnp.bfloat16),
    )(agg0, y0, dinv, w1blk, b1cat)


def _tc_mid3(agg2, ycat, dinv, kc2w, kc2b, ec2w, ec2b,
             kfc_w, kfc_b, efc_w, efc_b):
    def f(a_ref, ycat_ref, dinv_ref, kw2_ref, kb2_ref,
          ew2_ref, eb2_ref, kw_ref, kb_ref, ew_ref, eb_ref, kd_ref, ed_ref):
        dinv = dinv_ref[:N]
        g1k = dinv * (a_ref[:N, :HID].astype(jnp.float32)
                      + ycat_ref[:N, :HID].astype(jnp.float32))
        g1e = dinv * (a_ref[NACC:NACC + N, :HID].astype(jnp.float32)
                      + ycat_ref[NACC:NACC + N, :HID].astype(jnp.float32))
        h2k = jax.nn.relu(
            jnp.dot(g1k, kw2_ref[...], preferred_element_type=jnp.float32)
            + kb2_ref[...]
        )
        h2e = jax.nn.relu(
            jnp.dot(g1e, ew2_ref[...], preferred_element_type=jnp.float32)
            + eb2_ref[...]
        )
        pk = jnp.mean(h2k, axis=0, keepdims=True)
        pe = jnp.mean(h2e, axis=0, keepdims=True)
        kd_ref[...] = jax.nn.sigmoid(
            jnp.dot(pk, kw_ref[...], preferred_element_type=jnp.float32)
            + kb_ref[...]
        )
        ed_ref[...] = jax.nn.sigmoid(
            jnp.dot(pe, ew_ref[...], preferred_element_type=jnp.float32)
            + eb_ref[...]
        )

    return pl.pallas_call(
        f,
        out_shape=(
            jax.ShapeDtypeStruct((1, K), jnp.float32),
            jax.ShapeDtypeStruct((1, 1), jnp.float32),
        ),
    )(agg2, ycat, dinv, kc2w, kc2b, ec2w, ec2b,
      kfc_w, kfc_b, efc_w, efc_b)


def _tc_prednet(se_raw, kd, ed, kp, w1, b1, w2, b2, w3, b3):
    BN = 1024

    def f(se_ref, kd_ref, ed_ref, kp_ref, w1_ref, b1_ref, w2_ref, b2_ref,
          w3_ref, b3_ref, o_ref):
        x = ed_ref[0, 0] * (jax.nn.sigmoid(se_ref[...]) - kd_ref[...]) * kp_ref[...]
        z = jax.nn.sigmoid(
            jnp.dot(x, jnp.abs(w1_ref[...]), preferred_element_type=jnp.float32)
            + b1_ref[...]
        )
        z = jax.nn.sigmoid(
            jnp.dot(z, jnp.abs(w2_ref[...]), preferred_element_type=jnp.float32)
            + b2_ref[...]
        )
        o_ref[...] = jax.nn.sigmoid(
            jnp.dot(z, jnp.abs(w3_ref[...]), preferred_element_type=jnp.float32)
            + b3_ref[...]
        )

    return pl.pallas_call(
        f,
        grid=(B // BN,),
        in_specs=[
            pl.BlockSpec((BN, K), lambda i: (i, 0)),
            pl.BlockSpec((1, K), lambda i: (0, 0)),
            pl.BlockSpec((1, 1), lambda i: (0, 0)),
            pl.BlockSpec((BN, K), lambda i: (i, 0)),
            pl.BlockSpec((K, 512), lambda i: (0, 0)),
            pl.BlockSpec((1, 512), lambda i: (0, 0)),
            pl.BlockSpec((512, 256), lambda i: (0, 0)),
            pl.BlockSpec((1, 256), lambda i: (0, 0)),
            pl.BlockSpec((256, 1), lambda i: (0, 0)),
            pl.BlockSpec((1, 1), lambda i: (0, 0)),
        ],
        out_specs=pl.BlockSpec((BN, 1), lambda i: (i, 0)),
        out_shape=jax.ShapeDtypeStruct((B, 1), jnp.float32),
    )(se_raw, kd, ed, kp, w1, b1, w2, b2, w3, b3)


def kernel(stu_id, input_exercise, input_knowledge_point, teacher_x,
           teacher_edge_index, teacher_batch, student_emb, k_lin_W, k_lin_b,
           k_conv1_W, k_conv1_b, k_conv2_W, k_conv2_b, k_fc_W, k_fc_b,
           e_lin_W, e_lin_b, e_conv1_W, e_conv1_b, e_conv2_W, e_conv2_b,
           e_fc_W, e_fc_b, pf1_W, pf1_b, pf2_W, pf2_b, pf3_W, pf3_b):
    src = teacher_edge_index[0]
    dst = teacher_edge_index[1]

    pad_rows = jnp.asarray(_PAD_ROWS)
    src2d = jnp.concatenate([src.reshape(E // 128, 128), pad_rows])
    dst2d = jnp.concatenate([dst.reshape(E // 128, 128), pad_rows])
    srcb = jnp.concatenate([src2d, src2d + NACC])

    zeros32b = jnp.zeros((NACC, 32), jnp.bfloat16)
    zeros48b = jnp.zeros((NACC, 48), jnp.bfloat16)
    zeros_col = jnp.zeros((NACC, 8), jnp.float32)
    ones_col = jnp.ones((128, 8), jnp.float32)

    wc = jnp.concatenate([k_lin_W, e_lin_W], axis=1)
    bc = jnp.concatenate([k_lin_b, e_lin_b]).reshape(1, 32)
    z1640 = jnp.zeros((16, 40), jnp.float32)
    w1blk = jnp.concatenate(
        [jnp.concatenate([k_conv1_W, z1640], axis=1),
         jnp.concatenate([z1640, e_conv1_W], axis=1)], axis=0)
    b1cat = jnp.concatenate([k_conv1_b, e_conv1_b]).reshape(1, 80)

    se_raw = _sc_embed_gather(student_emb, stu_id)
    degp = _sc_degree(dst2d, ones_col, zeros_col)
    h0 = _tc_big_matmul(teacher_x, wc, bc)
    dinv, y0 = _tc_mid1(degp, h0)
    agg0 = _sc_aggregate(y0, src2d, dst2d, zeros32b, 32)
    ycat = _tc_mid2(agg0, y0, dinv, w1blk, b1cat)
    agg2 = _sc_aggregate_split(ycat, srcb, dst2d, zeros48b)
    kd, ed = _tc_mid3(agg2, ycat, dinv,
                      k_conv2_W, k_conv2_b.reshape(1, HID),
                      e_conv2_W, e_conv2_b.reshape(1, HID),
                      k_fc_W, k_fc_b.reshape(1, 1 * K),
                      e_fc_W, e_fc_b.reshape(1, 1))
    out = _tc_prednet(se_raw, kd, ed, input_knowledge_point,
                      pf1_W, pf1_b.reshape(1, 512),
                      pf2_W, pf2_b.reshape(1, 256),
                      pf3_W, pf3_b.reshape(1, 1))
    return out.reshape(-1)


# speedup vs baseline: 46.4277x; 1.1209x over previous
import functools

import numpy as np

import jax
import jax.numpy as jnp
from jax import lax
from jax.experimental import pallas as pl
from jax.experimental.pallas import tpu as pltpu
from jax.experimental.pallas import tpu_sc as plsc

N = 10000
E = 320000
B = 4096
K = 128
IN_FEAT = 4096
HID = 40
NACC = 10240
EPAD = 327680
NPADROWS = 240
NW = 32
RPT = NACC // 16

_MESH = dict(core_axis_name="c", subcore_axis_name="s")
_SC_PARAMS = pltpu.CompilerParams(use_tc_tiling_on_sc=False)

_PAD_ROWS = np.asarray(
    N + (np.arange(60 * 128, dtype=np.int32) % NPADROWS)).reshape(60, 128)


def _sc_degree(dst2d, ones_col, zeros_col):
    @functools.partial(
        pl.kernel,
        out_type=jax.ShapeDtypeStruct((2 * NACC, 8), jnp.float32),
        mesh=plsc.VectorSubcoreMesh(**_MESH),
        compiler_params=_SC_PARAMS,
        scratch_types=[
            pltpu.VMEM((80, 128), jnp.int32),
            pltpu.VMEM((128, 8), jnp.float32),
            pltpu.VMEM_SHARED((NACC, 8), jnp.float32),
            pltpu.SemaphoreType.DMA,
        ],
    )
    def deg_kernel(dst_h, ones_h, zeros_h, out_h, dst_v, ones_v, acc, sem):
        c = lax.axis_index("c")
        s = lax.axis_index("s")
        wid = s * 2 + c
        r0 = s * RPT
        pltpu.sync_copy(zeros_h.at[pl.ds(r0, RPT)], acc.at[pl.ds(r0, RPT)])
        pltpu.sync_copy(ones_h, ones_v)
        pltpu.sync_copy(dst_h.at[pl.ds(wid * 80, 80)], dst_v)
        plsc.subcore_barrier()

        lag = 8
        sca = {}
        for g in range(80):
            if g >= lag:
                sca[g - lag].wait()
            sca[g] = pltpu.async_copy(ones_v, acc.at[dst_v.at[g]], sem, add=True)
        for g in range(80 - lag, 80):
            sca[g].wait()
        plsc.subcore_barrier()
        pltpu.sync_copy(acc.at[pl.ds(r0, RPT)], out_h.at[pl.ds(c * NACC + r0, RPT)])

    return deg_kernel(dst2d, ones_col, zeros_col)


def _sc_aggregate(y, src2d, dst2d, zeros, w, dtype=jnp.bfloat16):
    @functools.partial(
        pl.kernel,
        out_type=jax.ShapeDtypeStruct((2 * NACC, w), dtype),
        mesh=plsc.VectorSubcoreMesh(**_MESH),
        compiler_params=_SC_PARAMS,
        scratch_types=[
            pltpu.VMEM((80, 128), jnp.int32),
            pltpu.VMEM((80, 128), jnp.int32),
            pltpu.VMEM((8, 128, w), dtype),
            pltpu.VMEM_SHARED((NACC, w), dtype),
            pltpu.SemaphoreType.DMA,
            pltpu.SemaphoreType.DMA,
        ],
    )
    def agg_kernel(y_h, src_h, dst_h, zeros_h, out_h, src_v, dst_v, msg_v, acc,
                   gsem, ssem):
        c = lax.axis_index("c")
        s = lax.axis_index("s")
        wid = s * 2 + c
        r0 = s * RPT
        pltpu.sync_copy(zeros_h.at[pl.ds(r0, RPT)], acc.at[pl.ds(r0, RPT)])
        pltpu.sync_copy(src_h.at[pl.ds(wid * 80, 80)], src_v)
        pltpu.sync_copy(dst_h.at[pl.ds(wid * 80, 80)], dst_v)
        plsc.subcore_barrier()

        gat, sca = {}, {}
        for g in range(6):
            gat[g] = pltpu.async_copy(y_h.at[src_v.at[g]], msg_v.at[g % 8], gsem)
        for g in range(80):
            if g >= 2:
                sca[g - 2].wait()
            if g + 6 < 80:
                gat[g + 6] = pltpu.async_copy(
                    y_h.at[src_v.at[g + 6]], msg_v.at[(g + 6) % 8], gsem)
            gat[g].wait()
            sca[g] = pltpu.async_copy(
                msg_v.at[g % 8], acc.at[dst_v.at[g]], ssem, add=True)
        sca[78].wait()
        sca[79].wait()
        plsc.subcore_barrier()
        pltpu.sync_copy(acc.at[pl.ds(r0, RPT)], out_h.at[pl.ds(c * NACC + r0, RPT)])

    return agg_kernel(y, src2d, dst2d, zeros)


def _sc_aggregate_split(ycat, srcb, dst2d, zeros):
    @functools.partial(
        pl.kernel,
        out_type=jax.ShapeDtypeStruct((2 * NACC, 48), jnp.bfloat16),
        mesh=plsc.VectorSubcoreMesh(**_MESH),
        compiler_params=_SC_PARAMS,
        scratch_types=[
            pltpu.VMEM((80, 128), jnp.int32),
            pltpu.VMEM((80, 128), jnp.int32),
            pltpu.VMEM((8, 128, 48), jnp.bfloat16),
            pltpu.VMEM_SHARED((NACC, 48), jnp.bfloat16),
            pltpu.SemaphoreType.DMA,
            pltpu.SemaphoreType.DMA,
        ],
    )
    def agg2_kernel(y_h, src_h, dst_h, zeros_h, out_h, src_v, dst_v, msg_v,
                    acc, gsem, ssem):
        c = lax.axis_index("c")
        s = lax.axis_index("s")
        r0 = s * RPT
        pltpu.sync_copy(zeros_h.at[pl.ds(r0, RPT)], acc.at[pl.ds(r0, RPT)])
        plsc.subcore_barrier()

        def one_pass(p, carry):
            srow = c * 2560 + s * 160 + p * 80
            drow = s * 160 + p * 80
            pltpu.sync_copy(src_h.at[pl.ds(srow, 80)], src_v)
            pltpu.sync_copy(dst_h.at[pl.ds(drow, 80)], dst_v)
            gat, sca = {}, {}
            for g in range(6):
                gat[g] = pltpu.async_copy(
                    y_h.at[src_v.at[g]], msg_v.at[g % 8], gsem)
            for g in range(80):
                if g >= 2:
                    sca[g - 2].wait()
                if g + 6 < 80:
                    gat[g + 6] = pltpu.async_copy(
                        y_h.at[src_v.at[g + 6]], msg_v.at[(g + 6) % 8], gsem)
                gat[g].wait()
                sca[g] = pltpu.async_copy(
                    msg_v.at[g % 8], acc.at[dst_v.at[g]], ssem, add=True)
            sca[78].wait()
            sca[79].wait()
            return carry

        lax.fori_loop(0, 2, one_pass, 0)
        plsc.subcore_barrier()
        pltpu.sync_copy(acc.at[pl.ds(r0, RPT)], out_h.at[pl.ds(c * NACC + r0, RPT)])

    return agg2_kernel(ycat, srcb, dst2d, zeros)


def _sc_embed_gather(table, idx):
    bpw = B // NW

    @functools.partial(
        pl.kernel,
        out_type=jax.ShapeDtypeStruct((B, K), jnp.float32),
        mesh=plsc.VectorSubcoreMesh(**_MESH),
        compiler_params=_SC_PARAMS,
        scratch_types=[
            pltpu.VMEM((bpw,), jnp.int32),
            pltpu.VMEM((bpw, K), jnp.float32),
            pltpu.SemaphoreType.DMA,
        ],
    )
    def gather_kernel(table_h, idx_h, out_h, idx_v, rows_v, sem):
        c = lax.axis_index("c")
        s = lax.axis_index("s")
        wid = s * 2 + c
        base = wid * bpw
        pltpu.sync_copy(idx_h.at[pl.ds(base, bpw)], idx_v)
        pltpu.async_copy(table_h.at[idx_v], rows_v, sem).wait()
        pltpu.sync_copy(rows_v, out_h.at[pl.ds(base, bpw)])

    return gather_kernel(table, idx)


def _tc_big_matmul(x, wc, bc):
    BN = 1000

    def f(x_ref, w_ref, b_ref, o_ref):
        o_ref[...] = (
            jnp.dot(x_ref[...], w_ref[...], preferred_element_type=jnp.float32)
            + b_ref[...]
        )

    return pl.pallas_call(
        f,
        grid=(N // BN,),
        in_specs=[
            pl.BlockSpec((BN, IN_FEAT), lambda i: (i, 0)),
            pl.BlockSpec((IN_FEAT, 32), lambda i: (0, 0)),
            pl.BlockSpec((1, 32), lambda i: (0, 0)),
        ],
        out_specs=pl.BlockSpec((BN, 32), lambda i: (i, 0)),
        out_shape=jax.ShapeDtypeStruct((N, 32), jnp.float32),
    )(x, wc, bc)


def _tc_mid1(degp, h0):
    def f(degp_ref, h0_ref, dinv_ref, y0_ref):
        deg = (degp_ref[:NACC, :1] + degp_ref[NACC:, :1]) + 1.0
        dinv = lax.rsqrt(deg)
        dinv_ref[...] = dinv
        y0_ref[:N] = (dinv[:N] * h0_ref[...]).astype(jnp.bfloat16)
        y0_ref[N:] = jnp.zeros((NACC - N, 32), jnp.bfloat16)

    return pl.pallas_call(
        f,
        out_shape=(
            jax.ShapeDtypeStruct((NACC, 1), jnp.float32),
            jax.ShapeDtypeStruct((NACC, 32), jnp.bfloat16),
        ),
    )(degp, h0)


def _tc_mid2(agg0, y0, dinv, w1blk, b1cat):
    def f(a_ref, y0_ref, dinv_ref, w_ref, b_ref, ycat_ref):
        dinv = dinv_ref[:N]
        asum = (a_ref[:N].astype(jnp.float32)
                + a_ref[NACC:NACC + N].astype(jnp.float32)
                + y0_ref[:N].astype(jnp.float32))
        g0 = dinv * asum
        h1 = jax.nn.relu(
            jnp.dot(g0, w_ref[...], preferred_element_type=jnp.float32) + b_ref[...]
        )
        y1 = dinv * h1
        cpad = jnp.zeros((N, 8), jnp.float32)
        zpad = jnp.zeros((NACC - N, 48), jnp.bfloat16)
        ycat_ref[:N] = jnp.concatenate(
            [y1[:, :HID], cpad], axis=1).astype(jnp.bfloat16)
        ycat_ref[N:NACC] = zpad
        ycat_ref[NACC:NACC + N] = jnp.concatenate(
            [y1[:, HID:], cpad], axis=1).astype(jnp.bfloat16)
        ycat_ref[NACC + N:] = zpad

    return pl.pallas_call(
        f,
        out_shape=jax.ShapeDtypeStruct((2 * NACC, 48), jnp.bfloat16),
    )(agg0, y0, dinv, w1blk, b1cat)


def _tc_mid3(agg2, ycat, dinv, kc2w, kc2b, ec2w, ec2b,
             kfc_w, kfc_b, efc_w, efc_b):
    def f(a_ref, ycat_ref, dinv_ref, kw2_ref, kb2_ref,
          ew2_ref, eb2_ref, kw_ref, kb_ref, ew_ref, eb_ref, kd_ref, ed_ref):
        dinv = dinv_ref[:N]
        g1k = dinv * (a_ref[:N, :HID].astype(jnp.float32)
                      + ycat_ref[:N, :HID].astype(jnp.float32))
        g1e = dinv * (a_ref[NACC:NACC + N, :HID].astype(jnp.float32)
                      + ycat_ref[NACC:NACC + N, :HID].astype(jnp.float32))
        h2k = jax.nn.relu(
            jnp.dot(g1k, kw2_ref[...], preferred_element_type=jnp.float32)
            + kb2_ref[...]
        )
        h2e = jax.nn.relu(
            jnp.dot(g1e, ew2_ref[...], preferred_element_type=jnp.float32)
            + eb2_ref[...]
        )
        pk = jnp.mean(h2k, axis=0, keepdims=True)
        pe = jnp.mean(h2e, axis=0, keepdims=True)
        kd_ref[...] = jax.nn.sigmoid(
            jnp.dot(pk, kw_ref[...], preferred_element_type=jnp.float32)
            + kb_ref[...]
        )
        ed_ref[...] = jax.nn.sigmoid(
            jnp.dot(pe, ew_ref[...], preferred_element_type=jnp.float32)
            + eb_ref[...]
        )

    return pl.pallas_call(
        f,
        out_shape=(
            jax.ShapeDtypeStruct((1, K), jnp.float32),
            jax.ShapeDtypeStruct((1, 1), jnp.float32),
        ),
    )(agg2, ycat, dinv, kc2w, kc2b, ec2w, ec2b,
      kfc_w, kfc_b, efc_w, efc_b)


def _tc_prednet(se_raw, kd, ed, kp, w1, b1, w2, b2, w3, b3):
    BN = 1024

    def f(se_ref, kd_ref, ed_ref, kp_ref, w1_ref, b1_ref, w2_ref, b2_ref,
          w3_ref, b3_ref, o_ref):
        x = ed_ref[0, 0] * (jax.nn.sigmoid(se_ref[...]) - kd_ref[...]) * kp_ref[...]
        z = jax.nn.sigmoid(
            jnp.dot(x, jnp.abs(w1_ref[...]), preferred_element_type=jnp.float32)
            + b1_ref[...]
        )
        z = jax.nn.sigmoid(
            jnp.dot(z, jnp.abs(w2_ref[...]), preferred_element_type=jnp.float32)
            + b2_ref[...]
        )
        o_ref[...] = jax.nn.sigmoid(
            jnp.dot(z, jnp.abs(w3_ref[...]), preferred_element_type=jnp.float32)
            + b3_ref[...]
        )

    return pl.pallas_call(
        f,
        grid=(B // BN,),
        in_specs=[
            pl.BlockSpec((BN, K), lambda i: (i, 0)),
            pl.BlockSpec((1, K), lambda i: (0, 0)),
            pl.BlockSpec((1, 1), lambda i: (0, 0)),
            pl.BlockSpec((BN, K), lambda i: (i, 0)),
            pl.BlockSpec((K, 512), lambda i: (0, 0)),
            pl.BlockSpec((1, 512), lambda i: (0, 0)),
            pl.BlockSpec((512, 256), lambda i: (0, 0)),
            pl.BlockSpec((1, 256), lambda i: (0, 0)),
            pl.BlockSpec((256, 1), lambda i: (0, 0)),
            pl.BlockSpec((1, 1), lambda i: (0, 0)),
        ],
        out_specs=pl.BlockSpec((BN, 1), lambda i: (i, 0)),
        out_shape=jax.ShapeDtypeStruct((B, 1), jnp.float32),
    )(se_raw, kd, ed, kp, w1, b1, w2, b2, w3, b3)


def kernel(stu_id, input_exercise, input_knowledge_point, teacher_x,
           teacher_edge_index, teacher_batch, student_emb, k_lin_W, k_lin_b,
           k_conv1_W, k_conv1_b, k_conv2_W, k_conv2_b, k_fc_W, k_fc_b,
           e_lin_W, e_lin_b, e_conv1_W, e_conv1_b, e_conv2_W, e_conv2_b,
           e_fc_W, e_fc_b, pf1_W, pf1_b, pf2_W, pf2_b, pf3_W, pf3_b):
    src = teacher_edge_index[0]
    dst = teacher_edge_index[1]

    pad_rows = jnp.asarray(_PAD_ROWS)
    src2d = jnp.concatenate([src.reshape(E // 128, 128), pad_rows])
    dst2d = jnp.concatenate([dst.reshape(E // 128, 128), pad_rows])
    srcb = jnp.concatenate([src2d, src2d + NACC])

    zeros32b = jnp.zeros((NACC, 32), jnp.bfloat16)
    zeros48b = jnp.zeros((NACC, 48), jnp.bfloat16)
    zeros_col = jnp.zeros((NACC, 8), jnp.float32)
    ones_col = jnp.ones((128, 8), jnp.float32)

    wc = jnp.concatenate([k_lin_W, e_lin_W], axis=1)
    bc = jnp.concatenate([k_lin_b, e_lin_b]).reshape(1, 32)
    z1640 = jnp.zeros((16, 40), jnp.float32)
    w1blk = jnp.concatenate(
        [jnp.concatenate([k_conv1_W, z1640], axis=1),
         jnp.concatenate([z1640, e_conv1_W], axis=1)], axis=0)
    b1cat = jnp.concatenate([k_conv1_b, e_conv1_b]).reshape(1, 80)

    se_raw = _sc_embed_gather(student_emb, stu_id)
    degp = _sc_degree(dst2d, ones_col, zeros_col)
    h0 = _tc_big_matmul(teacher_x, wc, bc)
    dinv, y0 = _tc_mid1(degp, h0)
    agg0 = _sc_aggregate(y0, src2d, dst2d, zeros32b, 32)
    ycat = _tc_mid2(agg0, y0, dinv, w1blk, b1cat)
    agg2 = _sc_aggregate_split(ycat, srcb, dst2d, zeros48b)
    kd, ed = _tc_mid3(agg2, ycat, dinv,
                      k_conv2_W, k_conv2_b.reshape(1, HID),
                      e_conv2_W, e_conv2_b.reshape(1, HID),
                      k_fc_W, k_fc_b.reshape(1, 1 * K),
                      e_fc_W, e_fc_b.reshape(1, 1))
    out = _tc_prednet(se_raw, kd, ed, input_knowledge_point,
                      pf1_W, pf1_b.reshape(1, 512),
                      pf2_W, pf2_b.reshape(1, 256),
                      pf3_W, pf3_b.reshape(1, 1))
    return out.reshape(-1)


# speedup vs baseline: 47.8605x; 1.0309x over previous
import functools

import numpy as np

import jax
import jax.numpy as jnp
from jax import lax
from jax.experimental import pallas as pl
from jax.experimental.pallas import tpu as pltpu
from jax.experimental.pallas import tpu_sc as plsc

N = 10000
E = 320000
B = 4096
K = 128
IN_FEAT = 4096
HID = 40
NACC = 10240
EPAD = 327680
NPADROWS = 240
NW = 32
RPT = NACC // 16

_MESH = dict(core_axis_name="c", subcore_axis_name="s")
_SC_PARAMS = pltpu.CompilerParams(use_tc_tiling_on_sc=False)

_PAD_ROWS = np.asarray(
    N + (np.arange(60 * 128, dtype=np.int32) % NPADROWS)).reshape(60, 128)


def _sc_degree(dst2d, ones_col, zeros_col):
    @functools.partial(
        pl.kernel,
        out_type=jax.ShapeDtypeStruct((2 * NACC, 8), jnp.float32),
        mesh=plsc.VectorSubcoreMesh(**_MESH),
        compiler_params=_SC_PARAMS,
        scratch_types=[
            pltpu.VMEM((80, 128), jnp.int32),
            pltpu.VMEM((128, 8), jnp.float32),
            pltpu.VMEM_SHARED((NACC, 8), jnp.float32),
            pltpu.SemaphoreType.DMA,
        ],
    )
    def deg_kernel(dst_h, ones_h, zeros_h, out_h, dst_v, ones_v, acc, sem):
        c = lax.axis_index("c")
        s = lax.axis_index("s")
        wid = s * 2 + c
        r0 = s * RPT
        pltpu.sync_copy(zeros_h.at[pl.ds(r0, RPT)], acc.at[pl.ds(r0, RPT)])
        pltpu.sync_copy(ones_h, ones_v)
        pltpu.sync_copy(dst_h.at[pl.ds(wid * 80, 80)], dst_v)
        plsc.subcore_barrier()

        lag = 8
        sca = {}
        for g in range(80):
            if g >= lag:
                sca[g - lag].wait()
            sca[g] = pltpu.async_copy(ones_v, acc.at[dst_v.at[g]], sem, add=True)
        for g in range(80 - lag, 80):
            sca[g].wait()
        plsc.subcore_barrier()
        pltpu.sync_copy(acc.at[pl.ds(r0, RPT)], out_h.at[pl.ds(c * NACC + r0, RPT)])

    return deg_kernel(dst2d, ones_col, zeros_col)


def _sc_aggregate(y, src2d, dst2d, zeros, w, dtype=jnp.bfloat16):
    @functools.partial(
        pl.kernel,
        out_type=jax.ShapeDtypeStruct((2 * NACC, w), dtype),
        mesh=plsc.VectorSubcoreMesh(**_MESH),
        compiler_params=_SC_PARAMS,
        scratch_types=[
            pltpu.VMEM((80, 128), jnp.int32),
            pltpu.VMEM((80, 128), jnp.int32),
            pltpu.VMEM((16, 128, w), dtype),
            pltpu.VMEM_SHARED((NACC, w), dtype),
            pltpu.SemaphoreType.DMA,
            pltpu.SemaphoreType.DMA,
        ],
    )
    def agg_kernel(y_h, src_h, dst_h, zeros_h, out_h, src_v, dst_v, msg_v, acc,
                   gsem, ssem):
        c = lax.axis_index("c")
        s = lax.axis_index("s")
        wid = s * 2 + c
        r0 = s * RPT
        pltpu.sync_copy(zeros_h.at[pl.ds(r0, RPT)], acc.at[pl.ds(r0, RPT)])
        pltpu.sync_copy(src_h.at[pl.ds(wid * 80, 80)], src_v)
        pltpu.sync_copy(dst_h.at[pl.ds(wid * 80, 80)], dst_v)
        plsc.subcore_barrier()

        gat, sca = {}, {}
        for g in range(10):
            gat[g] = pltpu.async_copy(y_h.at[src_v.at[g]], msg_v.at[g % 16], gsem)
        for g in range(80):
            if g >= 4:
                sca[g - 4].wait()
            if g + 10 < 80:
                gat[g + 10] = pltpu.async_copy(
                    y_h.at[src_v.at[g + 10]], msg_v.at[(g + 10) % 16], gsem)
            gat[g].wait()
            sca[g] = pltpu.async_copy(
                msg_v.at[g % 16], acc.at[dst_v.at[g]], ssem, add=True)
        for g in range(76, 80):
            sca[g].wait()
        plsc.subcore_barrier()
        pltpu.sync_copy(acc.at[pl.ds(r0, RPT)], out_h.at[pl.ds(c * NACC + r0, RPT)])

    return agg_kernel(y, src2d, dst2d, zeros)


def _sc_aggregate_split(ycat, srcb, dst2d, zeros):
    @functools.partial(
        pl.kernel,
        out_type=jax.ShapeDtypeStruct((2 * NACC, 48), jnp.bfloat16),
        mesh=plsc.VectorSubcoreMesh(**_MESH),
        compiler_params=_SC_PARAMS,
        scratch_types=[
            pltpu.VMEM((80, 128), jnp.int32),
            pltpu.VMEM((80, 128), jnp.int32),
            pltpu.VMEM((16, 128, 48), jnp.bfloat16),
            pltpu.VMEM_SHARED((NACC, 48), jnp.bfloat16),
            pltpu.SemaphoreType.DMA,
            pltpu.SemaphoreType.DMA,
        ],
    )
    def agg2_kernel(y_h, src_h, dst_h, zeros_h, out_h, src_v, dst_v, msg_v,
                    acc, gsem, ssem):
        c = lax.axis_index("c")
        s = lax.axis_index("s")
        r0 = s * RPT
        pltpu.sync_copy(zeros_h.at[pl.ds(r0, RPT)], acc.at[pl.ds(r0, RPT)])
        plsc.subcore_barrier()

        def one_pass(p, carry):
            srow = c * 2560 + s * 160 + p * 80
            drow = s * 160 + p * 80
            pltpu.sync_copy(src_h.at[pl.ds(srow, 80)], src_v)
            pltpu.sync_copy(dst_h.at[pl.ds(drow, 80)], dst_v)
            gat, sca = {}, {}
            for g in range(10):
                gat[g] = pltpu.async_copy(
                    y_h.at[src_v.at[g]], msg_v.at[g % 16], gsem)
            for g in range(80):
                if g >= 4:
                    sca[g - 4].wait()
                if g + 10 < 80:
                    gat[g + 10] = pltpu.async_copy(
                        y_h.at[src_v.at[g + 10]], msg_v.at[(g + 10) % 16], gsem)
                gat[g].wait()
                sca[g] = pltpu.async_copy(
                    msg_v.at[g % 16], acc.at[dst_v.at[g]], ssem, add=True)
            for g in range(76, 80):
                sca[g].wait()
            return carry

        lax.fori_loop(0, 2, one_pass, 0)
        plsc.subcore_barrier()
        pltpu.sync_copy(acc.at[pl.ds(r0, RPT)], out_h.at[pl.ds(c * NACC + r0, RPT)])

    return agg2_kernel(ycat, srcb, dst2d, zeros)


def _sc_embed_gather(table, idx):
    bpw = B // NW

    @functools.partial(
        pl.kernel,
        out_type=jax.ShapeDtypeStruct((B, K), jnp.float32),
        mesh=plsc.VectorSubcoreMesh(**_MESH),
        compiler_params=_SC_PARAMS,
        scratch_types=[
            pltpu.VMEM((bpw,), jnp.int32),
            pltpu.VMEM((bpw, K), jnp.float32),
            pltpu.SemaphoreType.DMA,
        ],
    )
    def gather_kernel(table_h, idx_h, out_h, idx_v, rows_v, sem):
        c = lax.axis_index("c")
        s = lax.axis_index("s")
        wid = s * 2 + c
        base = wid * bpw
        pltpu.sync_copy(idx_h.at[pl.ds(base, bpw)], idx_v)
        pltpu.async_copy(table_h.at[idx_v], rows_v, sem).wait()
        pltpu.sync_copy(rows_v, out_h.at[pl.ds(base, bpw)])

    return gather_kernel(table, idx)


def _tc_big_matmul(x, wc, bc):
    BN = 1000

    def f(x_ref, w_ref, b_ref, o_ref):
        o_ref[...] = (
            jnp.dot(x_ref[...], w_ref[...], preferred_element_type=jnp.float32)
            + b_ref[...]
        )

    return pl.pallas_call(
        f,
        grid=(N // BN,),
        in_specs=[
            pl.BlockSpec((BN, IN_FEAT), lambda i: (i, 0)),
            pl.BlockSpec((IN_FEAT, 32), lambda i: (0, 0)),
            pl.BlockSpec((1, 32), lambda i: (0, 0)),
        ],
        out_specs=pl.BlockSpec((BN, 32), lambda i: (i, 0)),
        out_shape=jax.ShapeDtypeStruct((N, 32), jnp.float32),
    )(x, wc, bc)


def _tc_mid1(degp, h0):
    def f(degp_ref, h0_ref, dinv_ref, y0_ref):
        deg = (degp_ref[:NACC, :1] + degp_ref[NACC:, :1]) + 1.0
        dinv = lax.rsqrt(deg)
        dinv_ref[...] = dinv
        y0_ref[:N] = (dinv[:N] * h0_ref[...]).astype(jnp.bfloat16)
        y0_ref[N:] = jnp.zeros((NACC - N, 32), jnp.bfloat16)

    return pl.pallas_call(
        f,
        out_shape=(
            jax.ShapeDtypeStruct((NACC, 1), jnp.float32),
            jax.ShapeDtypeStruct((NACC, 32), jnp.bfloat16),
        ),
    )(degp, h0)


def _tc_mid2(agg0, y0, dinv, w1blk, b1cat):
    def f(a_ref, y0_ref, dinv_ref, w_ref, b_ref, ycat_ref):
        dinv = dinv_ref[:N]
        asum = (a_ref[:N].astype(jnp.float32)
                + a_ref[NACC:NACC + N].astype(jnp.float32)
                + y0_ref[:N].astype(jnp.float32))
        g0 = dinv * asum
        h1 = jax.nn.relu(
            jnp.dot(g0, w_ref[...], preferred_element_type=jnp.float32) + b_ref[...]
        )
        y1 = dinv * h1
        cpad = jnp.zeros((N, 8), jnp.float32)
        zpad = jnp.zeros((NACC - N, 48), jnp.bfloat16)
        ycat_ref[:N] = jnp.concatenate(
            [y1[:, :HID], cpad], axis=1).astype(jnp.bfloat16)
        ycat_ref[N:NACC] = zpad
        ycat_ref[NACC:NACC + N] = jnp.concatenate(
            [y1[:, HID:], cpad], axis=1).astype(jnp.bfloat16)
        ycat_ref[NACC + N:] = zpad

    return pl.pallas_call(
        f,
        out_shape=jax.ShapeDtypeStruct((2 * NACC, 48), jnp.bfloat16),
    )(agg0, y0, dinv, w1blk, b1cat)


def _tc_mid3(agg2, ycat, dinv, kc2w, kc2b, ec2w, ec2b,
             kfc_w, kfc_b, efc_w, efc_b):
    def f(a_ref, ycat_ref, dinv_ref, kw2_ref, kb2_ref,
          ew2_ref, eb2_ref, kw_ref, kb_ref, ew_ref, eb_ref, kd_ref, ed_ref):
        dinv = dinv_ref[:N]
        g1k = dinv * (a_ref[:N, :HID].astype(jnp.float32)
                      + ycat_ref[:N, :HID].astype(jnp.float32))
        g1e = dinv * (a_ref[NACC:NACC + N, :HID].astype(jnp.float32)
                      + ycat_ref[NACC:NACC + N, :HID].astype(jnp.float32))
        h2k = jax.nn.relu(
            jnp.dot(g1k, kw2_ref[...], preferred_element_type=jnp.float32)
            + kb2_ref[...]
        )
        h2e = jax.nn.relu(
            jnp.dot(g1e, ew2_ref[...], preferred_element_type=jnp.float32)
            + eb2_ref[...]
        )
        pk = jnp.mean(h2k, axis=0, keepdims=True)
        pe = jnp.mean(h2e, axis=0, keepdims=True)
        kd_ref[...] = jax.nn.sigmoid(
            jnp.dot(pk, kw_ref[...], preferred_element_type=jnp.float32)
            + kb_ref[...]
        )
        ed_ref[...] = jax.nn.sigmoid(
            jnp.dot(pe, ew_ref[...], preferred_element_type=jnp.float32)
            + eb_ref[...]
        )

    return pl.pallas_call(
        f,
        out_shape=(
            jax.ShapeDtypeStruct((1, K), jnp.float32),
            jax.ShapeDtypeStruct((1, 1), jnp.float32),
        ),
    )(agg2, ycat, dinv, kc2w, kc2b, ec2w, ec2b,
      kfc_w, kfc_b, efc_w, efc_b)


def _tc_prednet(se_raw, kd, ed, kp, w1, b1, w2, b2, w3, b3):
    BN = 1024

    def f(se_ref, kd_ref, ed_ref, kp_ref, w1_ref, b1_ref, w2_ref, b2_ref,
          w3_ref, b3_ref, o_ref):
        x = ed_ref[0, 0] * (jax.nn.sigmoid(se_ref[...]) - kd_ref[...]) * kp_ref[...]
        z = jax.nn.sigmoid(
            jnp.dot(x, jnp.abs(w1_ref[...]), preferred_element_type=jnp.float32)
            + b1_ref[...]
        )
        z = jax.nn.sigmoid(
            jnp.dot(z, jnp.abs(w2_ref[...]), preferred_element_type=jnp.float32)
            + b2_ref[...]
        )
        o_ref[...] = jax.nn.sigmoid(
            jnp.dot(z, jnp.abs(w3_ref[...]), preferred_element_type=jnp.float32)
            + b3_ref[...]
        )

    return pl.pallas_call(
        f,
        grid=(B // BN,),
        in_specs=[
            pl.BlockSpec((BN, K), lambda i: (i, 0)),
            pl.BlockSpec((1, K), lambda i: (0, 0)),
            pl.BlockSpec((1, 1), lambda i: (0, 0)),
            pl.BlockSpec((BN, K), lambda i: (i, 0)),
            pl.BlockSpec((K, 512), lambda i: (0, 0)),
            pl.BlockSpec((1, 512), lambda i: (0, 0)),
            pl.BlockSpec((512, 256), lambda i: (0, 0)),
            pl.BlockSpec((1, 256), lambda i: (0, 0)),
            pl.BlockSpec((256, 1), lambda i: (0, 0)),
            pl.BlockSpec((1, 1), lambda i: (0, 0)),
        ],
        out_specs=pl.BlockSpec((BN, 1), lambda i: (i, 0)),
        out_shape=jax.ShapeDtypeStruct((B, 1), jnp.float32),
    )(se_raw, kd, ed, kp, w1, b1, w2, b2, w3, b3)


def kernel(stu_id, input_exercise, input_knowledge_point, teacher_x,
           teacher_edge_index, teacher_batch, student_emb, k_lin_W, k_lin_b,
           k_conv1_W, k_conv1_b, k_conv2_W, k_conv2_b, k_fc_W, k_fc_b,
           e_lin_W, e_lin_b, e_conv1_W, e_conv1_b, e_conv2_W, e_conv2_b,
           e_fc_W, e_fc_b, pf1_W, pf1_b, pf2_W, pf2_b, pf3_W, pf3_b):
    src = teacher_edge_index[0]
    dst = teacher_edge_index[1]

    pad_rows = jnp.asarray(_PAD_ROWS)
    src2d = jnp.concatenate([src.reshape(E // 128, 128), pad_rows])
    dst2d = jnp.concatenate([dst.reshape(E // 128, 128), pad_rows])
    srcb = jnp.concatenate([src2d, src2d + NACC])

    zeros32b = jnp.zeros((NACC, 32), jnp.bfloat16)
    zeros48b = jnp.zeros((NACC, 48), jnp.bfloat16)
    zeros_col = jnp.zeros((NACC, 8), jnp.float32)
    ones_col = jnp.ones((128, 8), jnp.float32)

    wc = jnp.concatenate([k_lin_W, e_lin_W], axis=1)
    bc = jnp.concatenate([k_lin_b, e_lin_b]).reshape(1, 32)
    z1640 = jnp.zeros((16, 40), jnp.float32)
    w1blk = jnp.concatenate(
        [jnp.concatenate([k_conv1_W, z1640], axis=1),
         jnp.concatenate([z1640, e_conv1_W], axis=1)], axis=0)
    b1cat = jnp.concatenate([k_conv1_b, e_conv1_b]).reshape(1, 80)

    se_raw = _sc_embed_gather(student_emb, stu_id)
    degp = _sc_degree(dst2d, ones_col, zeros_col)
    h0 = _tc_big_matmul(teacher_x, wc, bc)
    dinv, y0 = _tc_mid1(degp, h0)
    agg0 = _sc_aggregate(y0, src2d, dst2d, zeros32b, 32)
    ycat = _tc_mid2(agg0, y0, dinv, w1blk, b1cat)
    agg2 = _sc_aggregate_split(ycat, srcb, dst2d, zeros48b)
    kd, ed = _tc_mid3(agg2, ycat, dinv,
                      k_conv2_W, k_conv2_b.reshape(1, HID),
                      e_conv2_W, e_conv2_b.reshape(1, HID),
                      k_fc_W, k_fc_b.reshape(1, 1 * K),
                      e_fc_W, e_fc_b.reshape(1, 1))
    out = _tc_prednet(se_raw, kd, ed, input_knowledge_point,
                      pf1_W, pf1_b.reshape(1, 512),
                      pf2_W, pf2_b.reshape(1, 256),
                      pf3_W, pf3_b.reshape(1, 1))
    return out.reshape(-1)


# speedup vs baseline: 49.7080x; 1.0386x over previous
import functools

import numpy as np

import jax
import jax.numpy as jnp
from jax import lax
from jax.experimental import pallas as pl
from jax.experimental.pallas import tpu as pltpu
from jax.experimental.pallas import tpu_sc as plsc

N = 10000
E = 320000
B = 4096
K = 128
IN_FEAT = 4096
HID = 40
NACC = 10240
EPAD = 327680
NPADROWS = 240
NW = 32
RPT = NACC // 16

_MESH = dict(core_axis_name="c", subcore_axis_name="s")
_SC_PARAMS = pltpu.CompilerParams(use_tc_tiling_on_sc=False)

_PAD_ROWS = np.asarray(
    N + (np.arange(60 * 128, dtype=np.int32) % NPADROWS)).reshape(60, 128)


def _sc_degree(dst2d, ones_col, zeros_col):
    @functools.partial(
        pl.kernel,
        out_type=jax.ShapeDtypeStruct((2 * NACC, 8), jnp.float32),
        mesh=plsc.VectorSubcoreMesh(**_MESH),
        compiler_params=_SC_PARAMS,
        scratch_types=[
            pltpu.VMEM((80, 128), jnp.int32),
            pltpu.VMEM((128, 8), jnp.float32),
            pltpu.VMEM_SHARED((NACC, 8), jnp.float32),
            pltpu.SemaphoreType.DMA,
        ],
    )
    def deg_kernel(ei_h, pad_h, ones_h, zeros_h, out_h, dst_v, ones_v, acc, sem):
        c = lax.axis_index("c")
        s = lax.axis_index("s")
        wid = s * 2 + c
        r0 = s * RPT
        pltpu.sync_copy(zeros_h.at[pl.ds(r0, RPT)], acc.at[pl.ds(r0, RPT)])
        pltpu.sync_copy(ones_h, ones_v)

        @pl.when(wid < 31)
        def _():
            pltpu.sync_copy(ei_h.at[pl.ds(2500 + wid * 80, 80)], dst_v)

        @pl.when(wid == 31)
        def _():
            pltpu.sync_copy(ei_h.at[pl.ds(4980, 20)], dst_v.at[pl.ds(0, 20)])
            pltpu.sync_copy(pad_h, dst_v.at[pl.ds(20, 60)])

        plsc.subcore_barrier()

        lag = 8
        sca = {}
        for g in range(80):
            if g >= lag:
                sca[g - lag].wait()
            sca[g] = pltpu.async_copy(ones_v, acc.at[dst_v.at[g]], sem, add=True)
        for g in range(80 - lag, 80):
            sca[g].wait()
        plsc.subcore_barrier()
        pltpu.sync_copy(acc.at[pl.ds(r0, RPT)], out_h.at[pl.ds(c * NACC + r0, RPT)])

    return deg_kernel(dst2d, jnp.asarray(_PAD_ROWS), ones_col, zeros_col)


def _sc_aggregate(y, src2d, dst2d, zeros, w, dtype=jnp.bfloat16):
    @functools.partial(
        pl.kernel,
        out_type=jax.ShapeDtypeStruct((2 * NACC, w), dtype),
        mesh=plsc.VectorSubcoreMesh(**_MESH),
        compiler_params=_SC_PARAMS,
        scratch_types=[
            pltpu.VMEM((80, 128), jnp.int32),
            pltpu.VMEM((80, 128), jnp.int32),
            pltpu.VMEM((16, 128, w), dtype),
            pltpu.VMEM_SHARED((NACC, w), dtype),
            pltpu.SemaphoreType.DMA,
            pltpu.SemaphoreType.DMA,
        ],
    )
    def agg_kernel(y_h, ei_h, pad_h, zeros_h, out_h, src_v, dst_v, msg_v, acc,
                   gsem, ssem):
        c = lax.axis_index("c")
        s = lax.axis_index("s")
        wid = s * 2 + c
        r0 = s * RPT
        pltpu.sync_copy(zeros_h.at[pl.ds(r0, RPT)], acc.at[pl.ds(r0, RPT)])

        @pl.when(wid < 31)
        def _():
            pltpu.sync_copy(ei_h.at[pl.ds(wid * 80, 80)], src_v)
            pltpu.sync_copy(ei_h.at[pl.ds(2500 + wid * 80, 80)], dst_v)

        @pl.when(wid == 31)
        def _():
            pltpu.sync_copy(ei_h.at[pl.ds(2480, 20)], src_v.at[pl.ds(0, 20)])
            pltpu.sync_copy(pad_h, src_v.at[pl.ds(20, 60)])
            pltpu.sync_copy(ei_h.at[pl.ds(4980, 20)], dst_v.at[pl.ds(0, 20)])
            pltpu.sync_copy(pad_h, dst_v.at[pl.ds(20, 60)])

        plsc.subcore_barrier()

        gat, sca = {}, {}
        for g in range(10):
            gat[g] = pltpu.async_copy(y_h.at[src_v.at[g]], msg_v.at[g % 16], gsem)
        for g in range(80):
            if g >= 4:
                sca[g - 4].wait()
            if g + 10 < 80:
                gat[g + 10] = pltpu.async_copy(
                    y_h.at[src_v.at[g + 10]], msg_v.at[(g + 10) % 16], gsem)
            gat[g].wait()
            sca[g] = pltpu.async_copy(
                msg_v.at[g % 16], acc.at[dst_v.at[g]], ssem, add=True)
        for g in range(76, 80):
            sca[g].wait()
        plsc.subcore_barrier()
        pltpu.sync_copy(acc.at[pl.ds(r0, RPT)], out_h.at[pl.ds(c * NACC + r0, RPT)])

    return agg_kernel(y, src2d, dst2d, zeros)


def _sc_aggregate_split(yk, ye, ei2d, pad, zeros):
    @functools.partial(
        pl.kernel,
        out_type=jax.ShapeDtypeStruct((2 * NACC, 48), jnp.bfloat16),
        mesh=plsc.VectorSubcoreMesh(**_MESH),
        compiler_params=_SC_PARAMS,
        scratch_types=[
            pltpu.VMEM((80, 128), jnp.int32),
            pltpu.VMEM((80, 128), jnp.int32),
            pltpu.VMEM((16, 128, 48), jnp.bfloat16),
            pltpu.VMEM_SHARED((NACC, 48), jnp.bfloat16),
            pltpu.SemaphoreType.DMA,
            pltpu.SemaphoreType.DMA,
        ],
    )
    def agg2_kernel(yk_h, ye_h, ei_h, pad_h, zeros_h, out_h, src_v, dst_v,
                    msg_v, acc, gsem, ssem):
        c = lax.axis_index("c")
        s = lax.axis_index("s")
        r0 = s * RPT
        pltpu.sync_copy(zeros_h.at[pl.ds(r0, RPT)], acc.at[pl.ds(r0, RPT)])
        plsc.subcore_barrier()

        def issue_gat(g):
            @pl.when(c == 0)
            def _():
                pltpu.async_copy(yk_h.at[src_v.at[g]], msg_v.at[g % 16], gsem)

            @pl.when(c == 1)
            def _():
                pltpu.async_copy(ye_h.at[src_v.at[g]], msg_v.at[g % 16], gsem)

        def wait_gat(g):
            pltpu.make_async_copy(
                yk_h.at[src_v.at[g]], msg_v.at[g % 16], gsem).wait()

        def one_pass(p, carry):
            g0 = s * 160 + p * 80
            last = jnp.logical_and(s == 15, p == 1)

            @pl.when(jnp.logical_not(last))
            def _():
                pltpu.sync_copy(ei_h.at[pl.ds(g0, 80)], src_v)
                pltpu.sync_copy(ei_h.at[pl.ds(2500 + g0, 80)], dst_v)

            @pl.when(last)
            def _():
                pltpu.sync_copy(ei_h.at[pl.ds(2480, 20)], src_v.at[pl.ds(0, 20)])
                pltpu.sync_copy(pad_h, src_v.at[pl.ds(20, 60)])
                pltpu.sync_copy(ei_h.at[pl.ds(4980, 20)], dst_v.at[pl.ds(0, 20)])
                pltpu.sync_copy(pad_h, dst_v.at[pl.ds(20, 60)])

            sca = {}
            for g in range(10):
                issue_gat(g)
            for g in range(80):
                if g >= 4:
                    sca[g - 4].wait()
                if g + 10 < 80:
                    issue_gat(g + 10)
                wait_gat(g)
                sca[g] = pltpu.async_copy(
                    msg_v.at[g % 16], acc.at[dst_v.at[g]], ssem, add=True)
            for g in range(76, 80):
                sca[g].wait()
            return carry

        lax.fori_loop(0, 2, one_pass, 0)
        plsc.subcore_barrier()
        pltpu.sync_copy(acc.at[pl.ds(r0, RPT)], out_h.at[pl.ds(c * NACC + r0, RPT)])

    return agg2_kernel(yk, ye, ei2d, pad, zeros)


def _sc_embed_gather(table, idx):
    bpw = B // NW

    @functools.partial(
        pl.kernel,
        out_type=jax.ShapeDtypeStruct((B, K), jnp.float32),
        mesh=plsc.VectorSubcoreMesh(**_MESH),
        compiler_params=_SC_PARAMS,
        scratch_types=[
            pltpu.VMEM((bpw,), jnp.int32),
            pltpu.VMEM((bpw, K), jnp.float32),
            pltpu.SemaphoreType.DMA,
        ],
    )
    def gather_kernel(table_h, idx_h, out_h, idx_v, rows_v, sem):
        c = lax.axis_index("c")
        s = lax.axis_index("s")
        wid = s * 2 + c
        base = wid * bpw
        pltpu.sync_copy(idx_h.at[pl.ds(base, bpw)], idx_v)
        pltpu.async_copy(table_h.at[idx_v], rows_v, sem).wait()
        pltpu.sync_copy(rows_v, out_h.at[pl.ds(base, bpw)])

    return gather_kernel(table, idx)


def _tc_big_matmul(x, wc, bc):
    BN = 1000

    def f(x_ref, w_ref, b_ref, o_ref):
        o_ref[...] = (
            jnp.dot(x_ref[...], w_ref[...], preferred_element_type=jnp.float32)
            + b_ref[...]
        )

    return pl.pallas_call(
        f,
        grid=(N // BN,),
        in_specs=[
            pl.BlockSpec((BN, IN_FEAT), lambda i: (i, 0)),
            pl.BlockSpec((IN_FEAT, 32), lambda i: (0, 0)),
            pl.BlockSpec((1, 32), lambda i: (0, 0)),
        ],
        out_specs=pl.BlockSpec((BN, 32), lambda i: (i, 0)),
        out_shape=jax.ShapeDtypeStruct((N, 32), jnp.float32),
    )(x, wc, bc)


def _tc_mid1(degp, h0):
    def f(degp_ref, h0_ref, dinv_ref, y0_ref):
        deg = (degp_ref[:NACC, :1] + degp_ref[NACC:, :1]) + 1.0
        dinv = lax.rsqrt(deg)
        dinv_ref[...] = dinv
        y0_ref[:N] = (dinv[:N] * h0_ref[...]).astype(jnp.bfloat16)
        y0_ref[N:] = jnp.zeros((NACC - N, 32), jnp.bfloat16)

    return pl.pallas_call(
        f,
        out_shape=(
            jax.ShapeDtypeStruct((NACC, 1), jnp.float32),
            jax.ShapeDtypeStruct((NACC, 32), jnp.bfloat16),
        ),
    )(degp, h0)


def _tc_mid2(agg0, y0, dinv, w1blk, b1cat):
    def f(a_ref, y0_ref, dinv_ref, w_ref, b_ref, yk_ref, ye_ref):
        dinv = dinv_ref[:N]
        asum = (a_ref[:N].astype(jnp.float32)
                + a_ref[NACC:NACC + N].astype(jnp.float32)
                + y0_ref[:N].astype(jnp.float32))
        g0 = dinv * asum
        h1 = jax.nn.relu(
            jnp.dot(g0, w_ref[...], preferred_element_type=jnp.float32) + b_ref[...]
        )
        y1 = dinv * h1
        cpad = jnp.zeros((N, 8), jnp.float32)
        zpad = jnp.zeros((NACC - N, 48), jnp.bfloat16)
        yk_ref[:N] = jnp.concatenate(
            [y1[:, :HID], cpad], axis=1).astype(jnp.bfloat16)
        yk_ref[N:] = zpad
        ye_ref[:N] = jnp.concatenate(
            [y1[:, HID:], cpad], axis=1).astype(jnp.bfloat16)
        ye_ref[N:] = zpad

    return pl.pallas_call(
        f,
        out_shape=(
            jax.ShapeDtypeStruct((NACC, 48), jnp.bfloat16),
            jax.ShapeDtypeStruct((NACC, 48), jnp.bfloat16),
        ),
    )(agg0, y0, dinv, w1blk, b1cat)


def _tc_mid3(agg2, yk, ye, dinv, kc2w, kc2b, ec2w, ec2b,
             kfc_w, kfc_b, efc_w, efc_b):
    def f(a_ref, yk_ref, ye_ref, dinv_ref, kw2_ref, kb2_ref,
          ew2_ref, eb2_ref, kw_ref, kb_ref, ew_ref, eb_ref, kd_ref, ed_ref):
        dinv = dinv_ref[:N]
        g1k = dinv * (a_ref[:N, :HID].astype(jnp.float32)
                      + yk_ref[:N, :HID].astype(jnp.float32))
        g1e = dinv * (a_ref[NACC:NACC + N, :HID].astype(jnp.float32)
                      + ye_ref[:N, :HID].astype(jnp.float32))
        h2k = jax.nn.relu(
            jnp.dot(g1k, kw2_ref[...], preferred_element_type=jnp.float32)
            + kb2_ref[...]
        )
        h2e = jax.nn.relu(
            jnp.dot(g1e, ew2_ref[...], preferred_element_type=jnp.float32)
            + eb2_ref[...]
        )
        pk = jnp.mean(h2k, axis=0, keepdims=True)
        pe = jnp.mean(h2e, axis=0, keepdims=True)
        kd_ref[...] = jax.nn.sigmoid(
            jnp.dot(pk, kw_ref[...], preferred_element_type=jnp.float32)
            + kb_ref[...]
        )
        ed_ref[...] = jax.nn.sigmoid(
            jnp.dot(pe, ew_ref[...], preferred_element_type=jnp.float32)
            + eb_ref[...]
        )

    return pl.pallas_call(
        f,
        out_shape=(
            jax.ShapeDtypeStruct((1, K), jnp.float32),
            jax.ShapeDtypeStruct((1, 1), jnp.float32),
        ),
    )(agg2, yk, ye, dinv, kc2w, kc2b, ec2w, ec2b,
      kfc_w, kfc_b, efc_w, efc_b)


def _tc_prednet(se_raw, kd, ed, kp, w1, b1, w2, b2, w3, b3):
    BN = 1024

    def f(se_ref, kd_ref, ed_ref, kp_ref, w1_ref, b1_ref, w2_ref, b2_ref,
          w3_ref, b3_ref, o_ref):
        bf = jnp.bfloat16
        x = ed_ref[0, 0] * (jax.nn.sigmoid(se_ref[...]) - kd_ref[...]) * kp_ref[...]
        z = jax.nn.sigmoid(
            jnp.dot(x.astype(bf), jnp.abs(w1_ref[...]).astype(bf),
                    preferred_element_type=jnp.float32) + b1_ref[...]
        )
        z = jax.nn.sigmoid(
            jnp.dot(z.astype(bf), jnp.abs(w2_ref[...]).astype(bf),
                    preferred_element_type=jnp.float32) + b2_ref[...]
        )
        o_ref[...] = jax.nn.sigmoid(
            jnp.dot(z.astype(bf), jnp.abs(w3_ref[...]).astype(bf),
                    preferred_element_type=jnp.float32) + b3_ref[...]
        )

    return pl.pallas_call(
        f,
        grid=(B // BN,),
        in_specs=[
            pl.BlockSpec((BN, K), lambda i: (i, 0)),
            pl.BlockSpec((1, K), lambda i: (0, 0)),
            pl.BlockSpec((1, 1), lambda i: (0, 0)),
            pl.BlockSpec((BN, K), lambda i: (i, 0)),
            pl.BlockSpec((K, 512), lambda i: (0, 0)),
            pl.BlockSpec((1, 512), lambda i: (0, 0)),
            pl.BlockSpec((512, 256), lambda i: (0, 0)),
            pl.BlockSpec((1, 256), lambda i: (0, 0)),
            pl.BlockSpec((256, 1), lambda i: (0, 0)),
            pl.BlockSpec((1, 1), lambda i: (0, 0)),
        ],
        out_specs=pl.BlockSpec((BN, 1), lambda i: (i, 0)),
        out_shape=jax.ShapeDtypeStruct((B, 1), jnp.float32),
    )(se_raw, kd, ed, kp, w1, b1, w2, b2, w3, b3)


def kernel(stu_id, input_exercise, input_knowledge_point, teacher_x,
           teacher_edge_index, teacher_batch, student_emb, k_lin_W, k_lin_b,
           k_conv1_W, k_conv1_b, k_conv2_W, k_conv2_b, k_fc_W, k_fc_b,
           e_lin_W, e_lin_b, e_conv1_W, e_conv1_b, e_conv2_W, e_conv2_b,
           e_fc_W, e_fc_b, pf1_W, pf1_b, pf2_W, pf2_b, pf3_W, pf3_b):
    ei2d = teacher_edge_index.reshape(2 * E // 128, 128)
    pad_rows = jnp.asarray(_PAD_ROWS)

    zeros32b = jnp.zeros((NACC, 32), jnp.bfloat16)
    zeros48b = jnp.zeros((NACC, 48), jnp.bfloat16)
    zeros_col = jnp.zeros((NACC, 8), jnp.float32)
    ones_col = jnp.ones((128, 8), jnp.float32)

    wc = jnp.concatenate([k_lin_W, e_lin_W], axis=1)
    bc = jnp.concatenate([k_lin_b, e_lin_b]).reshape(1, 32)
    z1640 = jnp.zeros((16, 40), jnp.float32)
    w1blk = jnp.concatenate(
        [jnp.concatenate([k_conv1_W, z1640], axis=1),
         jnp.concatenate([z1640, e_conv1_W], axis=1)], axis=0)
    b1cat = jnp.concatenate([k_conv1_b, e_conv1_b]).reshape(1, 80)

    se_raw = _sc_embed_gather(student_emb, stu_id)
    degp = _sc_degree(ei2d, ones_col, zeros_col)
    h0 = _tc_big_matmul(teacher_x, wc, bc)
    dinv, y0 = _tc_mid1(degp, h0)
    agg0 = _sc_aggregate(y0, ei2d, pad_rows, zeros32b, 32)
    yk, ye = _tc_mid2(agg0, y0, dinv, w1blk, b1cat)
    agg2 = _sc_aggregate_split(yk, ye, ei2d, pad_rows, zeros48b)
    kd, ed = _tc_mid3(agg2, yk, ye, dinv,
                      k_conv2_W, k_conv2_b.reshape(1, HID),
                      e_conv2_W, e_conv2_b.reshape(1, HID),
                      k_fc_W, k_fc_b.reshape(1, 1 * K),
                      e_fc_W, e_fc_b.reshape(1, 1))
    out = _tc_prednet(se_raw, kd, ed, input_knowledge_point,
                      pf1_W, pf1_b.reshape(1, 512),
                      pf2_W, pf2_b.reshape(1, 256),
                      pf3_W, pf3_b.reshape(1, 1))
    return out.reshape(-1)
